# baseline (speedup 1.0000x reference)
import functools

import jax
import jax.numpy as jnp
from jax import lax
from jax.experimental import pallas as pl
from jax.experimental.pallas import tpu as pltpu
from jax.experimental.pallas import tpu_sc as plsc

N = 50000
E = 400000
HID = 128
OUT = 16
NSUB = 16
NCORE = 2
BLK = 128
NBLK = E // BLK
CCH = 16
NCH = HID // CCH
ZR = 200
RPT = 3200
GG = 8

_MESH = plsc.VectorSubcoreMesh(core_axis_name="c", subcore_axis_name="s")


def _span(total, parts, i):
    base = total // parts
    rem = total % parts
    start = i * base + jnp.minimum(i, rem)
    cnt = base + jnp.where(i < rem, 1, 0).astype(jnp.int32)
    return start, cnt


def _fill(buf, rows, cols, value):
    v = jnp.full((16,), value, jnp.float32)

    def row(i, _):
        for c0 in range(0, cols, 16):
            buf[i, pl.ds(c0, 16)] = v
        return 0

    lax.fori_loop(0, rows, row, 0)


def _per_tile_rows(tid, fn):
    @pl.when(tid < NSUB - 1)
    def _():
        fn(tid * RPT, RPT // ZR)

    @pl.when(tid == NSUB - 1)
    def _():
        fn((NSUB - 1) * RPT, (N - (NSUB - 1) * RPT) // ZR)


def _zero_my_rows(acc_sp, zbuf, tid):
    def fn(row0, nch):
        for z in range(nch):
            pltpu.sync_copy(zbuf, acc_sp.at[pl.ds(row0 + z * ZR, ZR)])

    _per_tile_rows(tid, fn)


def _copy_my_rows(acc_sp, out_ref, tid):
    def fn(row0, nch):
        for z in range(nch):
            pltpu.sync_copy(acc_sp.at[pl.ds(row0 + z * ZR, ZR)],
                            out_ref.at[pl.ds(row0 + z * ZR, ZR)])

    _per_tile_rows(tid, fn)


def _accumulate(tab_h, src2_h, dst2_h, acc_sp, sg, dg, rb0, rb1, gsem0,
                gsem1, ssem0, ssem1, start, cnt):
    def fire_gather(idx, rb, sem, j0):
        return [pltpu.async_copy(tab_h.at[idx.at[j0 + j]],
                                 rb.at[pl.ds(j * BLK, BLK)], sem)
                for j in range(GG)]

    def fire_scatter(idx, rb, sem, j0):
        return [pltpu.async_copy(rb.at[pl.ds(j * BLK, BLK)],
                                 acc_sp.at[idx.at[j0 + j]], sem, add=True)
                for j in range(GG)]

    nsup = cnt // (2 * GG)

    def sup(s, _):
        base = start + s * 2 * GG
        pltpu.sync_copy(src2_h.at[pl.ds(base, 2 * GG)], sg)
        pltpu.sync_copy(dst2_h.at[pl.ds(base, 2 * GG)], dg)
        g0 = fire_gather(sg, rb0, gsem0, 0)
        g1 = fire_gather(sg, rb1, gsem1, GG)
        for d in g0:
            d.wait()
        s0 = fire_scatter(dg, rb0, ssem0, 0)
        for d in g1:
            d.wait()
        s1 = fire_scatter(dg, rb1, ssem1, GG)
        for d in s0:
            d.wait()
        for d in s1:
            d.wait()
        return 0

    lax.fori_loop(0, nsup, sup, 0)

    def tail(t, _):
        b = start + nsup * 2 * GG + t
        pltpu.sync_copy(src2_h.at[pl.ds(b, 1)], sg.at[pl.ds(0, 1)])
        pltpu.sync_copy(dst2_h.at[pl.ds(b, 1)], dg.at[pl.ds(0, 1)])
        pltpu.sync_copy(tab_h.at[sg.at[0]], rb0.at[pl.ds(0, BLK)])
        pltpu.sync_copy(rb0.at[pl.ds(0, BLK)], acc_sp.at[dg.at[0]], add=True)
        return 0

    lax.fori_loop(0, cnt - nsup * 2 * GG, tail, 0)


def _seg128(t8, src, dst):
    @functools.partial(
        pl.kernel,
        out_type=jax.ShapeDtypeStruct((NCH, N, CCH), jnp.float32),
        mesh=_MESH,
        compiler_params=pltpu.CompilerParams(use_tc_tiling_on_sc=False),
        scratch_types=[
            pltpu.VMEM_SHARED((N, CCH), jnp.float32),
            pltpu.VMEM((ZR, CCH), jnp.float32),
            pltpu.VMEM((2 * GG, BLK), jnp.int32),
            pltpu.VMEM((2 * GG, BLK), jnp.int32),
            pltpu.VMEM((GG * BLK, CCH), jnp.float32),
            pltpu.VMEM((GG * BLK, CCH), jnp.float32),
            pltpu.SemaphoreType.DMA,
            pltpu.SemaphoreType.DMA,
            pltpu.SemaphoreType.DMA,
            pltpu.SemaphoreType.DMA,
        ],
    )
    def k(t8_h, src_h, dst_h, out_h, acc_sp, zbuf, sg, dg, rb0, rb1, gsem0,
          gsem1, ssem0, ssem1):
        cid = lax.axis_index("c")
        tid = lax.axis_index("s")
        _fill(zbuf, ZR, CCH, 0.0)
        start, cnt = _span(NBLK, NSUB, tid)

        def chunk(j, _):
            ci = (NCH // NCORE) * cid + j
            _zero_my_rows(acc_sp, zbuf, tid)
            plsc.subcore_barrier()
            _accumulate(t8_h.at[ci], src_h, dst_h, acc_sp, sg, dg, rb0, rb1,
                        gsem0, gsem1, ssem0, ssem1, start, cnt)
            plsc.subcore_barrier()
            _copy_my_rows(acc_sp, out_h.at[ci], tid)
            plsc.subcore_barrier()
            return 0

        lax.fori_loop(0, NCH // NCORE, chunk, 0)

    return k(t8, src, dst)


def _seg16(z0, z1, src0, dst0, src1, dst1):
    @functools.partial(
        pl.kernel,
        out_type=jax.ShapeDtypeStruct((4, N, OUT), jnp.float32),
        mesh=_MESH,
        compiler_params=pltpu.CompilerParams(use_tc_tiling_on_sc=False),
        scratch_types=[
            pltpu.VMEM_SHARED((N, OUT), jnp.float32),
            pltpu.VMEM((ZR, OUT), jnp.float32),
            pltpu.VMEM((2 * GG, BLK), jnp.int32),
            pltpu.VMEM((2 * GG, BLK), jnp.int32),
            pltpu.VMEM((GG * BLK, OUT), jnp.float32),
            pltpu.VMEM((GG * BLK, OUT), jnp.float32),
            pltpu.SemaphoreType.DMA,
            pltpu.SemaphoreType.DMA,
            pltpu.SemaphoreType.DMA,
            pltpu.SemaphoreType.DMA,
        ],
    )
    def k(z0_h, z1_h, s0_h, d0_h, s1_h, d1_h, out_h, acc_sp, zbuf, sg, dg,
          rb0, rb1, gsem0, gsem1, ssem0, ssem1):
        cid = lax.axis_index("c")
        tid = lax.axis_index("s")
        _fill(zbuf, ZR, OUT, 0.0)
        cstart, ccnt = _span(NBLK, NCORE, cid)
        tstart, tcnt = _span(ccnt, NSUB, tid)
        start = cstart + tstart

        for rel, (z_h, s_h, d_h) in enumerate(((z0_h, s0_h, d0_h),
                                               (z1_h, s1_h, d1_h))):
            _zero_my_rows(acc_sp, zbuf, tid)
            plsc.subcore_barrier()
            _accumulate(z_h, s_h, d_h, acc_sp, sg, dg, rb0, rb1, gsem0,
                        gsem1, ssem0, ssem1, start, tcnt)
            plsc.subcore_barrier()

            @pl.when(cid == 0)
            def _():
                _copy_my_rows(acc_sp, out_h.at[2 * rel], tid)

            @pl.when(cid == 1)
            def _():
                _copy_my_rows(acc_sp, out_h.at[2 * rel + 1], tid)

            plsc.subcore_barrier()

    return k(z0, z1, src0, dst0, src1, dst1)


def _counts(dst_c, dst_cb, dst_f):
    @functools.partial(
        pl.kernel,
        out_type=(jax.ShapeDtypeStruct((N, OUT), jnp.float32),) * 3,
        mesh=_MESH,
        compiler_params=pltpu.CompilerParams(use_tc_tiling_on_sc=False),
        scratch_types=[
            pltpu.VMEM_SHARED((N, OUT), jnp.float32),
            pltpu.VMEM_SHARED((N, OUT), jnp.float32),
            pltpu.VMEM((ZR, OUT), jnp.float32),
            pltpu.VMEM((BLK, OUT), jnp.float32),
            pltpu.VMEM((2 * GG, BLK), jnp.int32),
            pltpu.VMEM((ZR, OUT), jnp.float32),
            pltpu.SemaphoreType.DMA,
        ],
    )
    def k(dc_h, dcb_h, df_h, qc_h, qcb_h, qf_h, acc0, acc1, zbuf, ones, dbuf,
          qbuf, ssem):
        cid = lax.axis_index("c")
        tid = lax.axis_index("s")
        _fill(zbuf, ZR, OUT, 0.0)
        _fill(ones, BLK, OUT, 1.0)
        start, cnt = _span(NBLK, NSUB, tid)

        def hist(d_h, acc):
            def group(base, nb):
                pltpu.sync_copy(d_h.at[pl.ds(base, nb)],
                                dbuf.at[pl.ds(0, nb)])
                sds = [pltpu.async_copy(ones, acc.at[dbuf.at[j]], ssem,
                                        add=True) for j in range(nb)]
                for d in sds:
                    d.wait()

            ngrp = cnt // (2 * GG)

            def body(g, _):
                group(start + g * 2 * GG, 2 * GG)
                return 0

            lax.fori_loop(0, ngrp, body, 0)

            def tailb(t, _):
                group(start + ngrp * 2 * GG + t, 1)
                return 0

            lax.fori_loop(0, cnt - ngrp * 2 * GG, tailb, 0)

        def finalize(acc, q_h):
            def fn(row0, nch):
                for z in range(nch):
                    r0 = row0 + z * ZR
                    pltpu.sync_copy(acc.at[pl.ds(r0, ZR)], qbuf)

                    def row(i, _):
                        v = qbuf[i, pl.ds(0, 16)]
                        r = 1.0 / jnp.maximum(v, 1.0)
                        qbuf[i, pl.ds(0, 16)] = jnp.where(
                            v > 0.5, r, jnp.zeros((16,), jnp.float32))
                        return 0

                    lax.fori_loop(0, ZR, row, 0)
                    pltpu.sync_copy(qbuf, q_h.at[pl.ds(r0, ZR)])

            _per_tile_rows(tid, fn)

        @pl.when(cid == 0)
        def _():
            _zero_my_rows(acc0, zbuf, tid)
            _zero_my_rows(acc1, zbuf, tid)
            plsc.subcore_barrier()
            hist(dc_h, acc0)
            hist(dcb_h, acc1)
            plsc.subcore_barrier()
            finalize(acc0, qc_h)
            finalize(acc1, qcb_h)

        @pl.when(cid == 1)
        def _():
            _zero_my_rows(acc0, zbuf, tid)
            plsc.subcore_barrier()
            hist(df_h, acc0)
            plsc.subcore_barrier()
            finalize(acc0, qf_h)

    return k(dst_c, dst_cb, dst_f)


RB = 1000
_GRID = N // RB


def _lrelu(x):
    return jnp.where(x >= 0, x, 0.01 * x)


def _chunk_mm(s, w):
    acc = jnp.dot(s[0], w[0:CCH, :], preferred_element_type=jnp.float32)
    for c in range(1, NCH):
        acc = acc + jnp.dot(s[c], w[c * CCH:(c + 1) * CCH, :],
                            preferred_element_type=jnp.float32)
    return acc


def _stage_b_body(sc_r, qc_r, scb_r, qcb_r, sf_r, qf_r, w0c_r, b0c_r, w0cb_r,
                  b0cb_r, w0f_r, b0f_r, w1cb_r, w1f_r, wlin_r, zi_r, zu_r):
    wlin = wlin_r[...]
    qc = qc_r[...][:, 0:1]
    gc = (qc > 0).astype(jnp.float32)
    item0 = qc * _chunk_mm(sc_r[...], w0c_r[...]) + gc * b0c_r[...]
    zi_r[...] = jnp.dot(_lrelu(item0),
                        jnp.dot(w1cb_r[...], wlin,
                                preferred_element_type=jnp.float32),
                        preferred_element_type=jnp.float32)
    qcb = qcb_r[...][:, 0:1]
    gcb = (qcb > 0).astype(jnp.float32)
    qf = qf_r[...][:, 0:1]
    gf = (qf > 0).astype(jnp.float32)
    user0 = (qcb * _chunk_mm(scb_r[...], w0cb_r[...]) + gcb * b0cb_r[...]
             + qf * _chunk_mm(sf_r[...], w0f_r[...]) + gf * b0f_r[...])
    zu_r[...] = jnp.dot(_lrelu(user0),
                        jnp.dot(w1f_r[...], wlin,
                                preferred_element_type=jnp.float32),
                        preferred_element_type=jnp.float32)


def _stage_b(sc, qc, scb, qcb, sf, qf, w0c, b0c, w0cb, b0cb, w0f, b0f, w1cb,
             w1f, wlin):
    s_spec = pl.BlockSpec((NCH, RB, CCH), lambda r: (0, r, 0))
    q_spec = pl.BlockSpec((RB, OUT), lambda r: (r, 0))
    w_spec = pl.BlockSpec((HID, HID), lambda r: (0, 0))
    b_spec = pl.BlockSpec((1, HID), lambda r: (0, 0))
    return pl.pallas_call(
        _stage_b_body,
        grid=(_GRID,),
        in_specs=[s_spec, q_spec, s_spec, q_spec, s_spec, q_spec,
                  w_spec, b_spec, w_spec, b_spec, w_spec, b_spec,
                  w_spec, w_spec, pl.BlockSpec((HID, OUT), lambda r: (0, 0))],
        out_specs=[q_spec, q_spec],
        out_shape=[jax.ShapeDtypeStruct((N, OUT), jnp.float32)] * 2,
    )(sc, qc, scb, qcb, sf, qf, w0c, b0c, w0cb, b0cb, w0f, b0f, w1cb, w1f,
      wlin)


def _stage_d_body(p_r, qcb_r, qf_r, b1cb_r, b1f_r, wlin_r, linb_r, out_r):
    p = p_r[...]
    qcb = qcb_r[...][:, 0:1]
    gcb = (qcb > 0).astype(jnp.float32)
    qf = qf_r[...][:, 0:1]
    gf = (qf > 0).astype(jnp.float32)
    wlin = wlin_r[...]
    bias = (gcb * jnp.dot(b1cb_r[...], wlin, preferred_element_type=jnp.float32)
            + gf * jnp.dot(b1f_r[...], wlin, preferred_element_type=jnp.float32)
            + linb_r[...])
    out_r[...] = qcb * (p[0] + p[1]) + qf * (p[2] + p[3]) + bias


def _stage_d(p, qcb, qf, b1cb, b1f, wlin, linb):
    q_spec = pl.BlockSpec((RB, OUT), lambda r: (r, 0))
    return pl.pallas_call(
        _stage_d_body,
        grid=(_GRID,),
        in_specs=[pl.BlockSpec((4, RB, OUT), lambda r: (0, r, 0)), q_spec,
                  q_spec, pl.BlockSpec((1, HID), lambda r: (0, 0)),
                  pl.BlockSpec((1, HID), lambda r: (0, 0)),
                  pl.BlockSpec((HID, OUT), lambda r: (0, 0)),
                  pl.BlockSpec((1, OUT), lambda r: (0, 0))],
        out_specs=q_spec,
        out_shape=jax.ShapeDtypeStruct((N, OUT), jnp.float32),
    )(p, qcb, qf, b1cb, b1f, wlin, linb)


def kernel(features, embed_item, edge_index_clicks, edge_index_clicked_by,
           edge_index_follows, W0_clicks, b0_clicks, W0_clicked_by,
           b0_clicked_by, W0_follows, b0_follows, W1_clicks, b1_clicks,
           W1_clicked_by, b1_clicked_by, W1_follows, b1_follows, lin_W,
           lin_b):
    i32 = jnp.int32
    r2 = lambda x: x.astype(i32).reshape(NBLK, BLK)
    sc_, dc_ = r2(edge_index_clicks[0]), r2(edge_index_clicks[1])
    scb, dcb = r2(edge_index_clicked_by[0]), r2(edge_index_clicked_by[1])
    sf_, df_ = r2(edge_index_follows[0]), r2(edge_index_follows[1])

    f8 = features.reshape(N, NCH, CCH).transpose(1, 0, 2)
    e8 = embed_item.reshape(N, NCH, CCH).transpose(1, 0, 2)

    qc, qcb, qf = _counts(dc_, dcb, df_)
    s_clicks = _seg128(f8, sc_, dc_)
    s_cb = _seg128(e8, scb, dcb)
    s_f = _seg128(f8, sf_, df_)

    zi, zu = _stage_b(s_clicks, qc, s_cb, qcb, s_f, qf,
                      W0_clicks, b0_clicks.reshape(1, HID),
                      W0_clicked_by, b0_clicked_by.reshape(1, HID),
                      W0_follows, b0_follows.reshape(1, HID),
                      W1_clicked_by, W1_follows, lin_W)

    p = _seg16(zi, zu, scb, dcb, sf_, df_)

    return _stage_d(p, qcb, qf, b1_clicked_by.reshape(1, HID),
                    b1_follows.reshape(1, HID), lin_W,
                    lin_b.reshape(1, OUT))


# speedup vs baseline: 1.0925x; 1.0925x over previous
import functools

import jax
import jax.numpy as jnp
from jax import lax
from jax.experimental import pallas as pl
from jax.experimental.pallas import tpu as pltpu
from jax.experimental.pallas import tpu_sc as plsc

N = 50000
E = 400000
HID = 128
OUT = 16
NSUB = 16
NCORE = 2
BLK = 128
NBLK = E // BLK
CCH = 32
NCH = HID // CCH
ZR = 80
RPT = 3200
GG = 3
GG16 = 8

_MESH = plsc.VectorSubcoreMesh(core_axis_name="c", subcore_axis_name="s")


def _span(total, parts, i):
    base = total // parts
    rem = total % parts
    start = i * base + jnp.minimum(i, rem)
    cnt = base + jnp.where(i < rem, 1, 0).astype(jnp.int32)
    return start, cnt


def _fill(buf, rows, cols, value):
    v = jnp.full((16,), value, jnp.float32)

    def row(i, _):
        for c0 in range(0, cols, 16):
            buf[i, pl.ds(c0, 16)] = v
        return 0

    lax.fori_loop(0, rows, row, 0)


def _per_tile_rows(tid, fn):
    @pl.when(tid < NSUB - 1)
    def _():
        fn(tid * RPT, RPT // ZR)

    @pl.when(tid == NSUB - 1)
    def _():
        fn((NSUB - 1) * RPT, (N - (NSUB - 1) * RPT) // ZR)


def _zero_my_rows(acc_sp, zbuf, tid):
    def fn(row0, nch):
        for z in range(nch):
            pltpu.sync_copy(zbuf, acc_sp.at[pl.ds(row0 + z * ZR, ZR)])

    _per_tile_rows(tid, fn)


def _copy_my_rows(acc_sp, out_ref, tid):
    def fn(row0, nch):
        for z in range(nch):
            pltpu.sync_copy(acc_sp.at[pl.ds(row0 + z * ZR, ZR)],
                            out_ref.at[pl.ds(row0 + z * ZR, ZR)])

    _per_tile_rows(tid, fn)


def _accumulate(gg, tab_h, src2_h, dst2_h, acc_sp, sg, dg, rb0, rb1, gsem0,
                gsem1, ssem0, ssem1, start, cnt):
    def fire_gather(idx, rb, sem, j0):
        return [pltpu.async_copy(tab_h.at[idx.at[j0 + j]],
                                 rb.at[pl.ds(j * BLK, BLK)], sem)
                for j in range(gg)]

    def fire_scatter(idx, rb, sem, j0):
        return [pltpu.async_copy(rb.at[pl.ds(j * BLK, BLK)],
                                 acc_sp.at[idx.at[j0 + j]], sem, add=True)
                for j in range(gg)]

    nsup = cnt // (2 * gg)

    def sup(s, _):
        base = start + s * 2 * gg
        pltpu.sync_copy(src2_h.at[pl.ds(base, 2 * gg)], sg)
        pltpu.sync_copy(dst2_h.at[pl.ds(base, 2 * gg)], dg)
        g0 = fire_gather(sg, rb0, gsem0, 0)
        g1 = fire_gather(sg, rb1, gsem1, gg)
        for d in g0:
            d.wait()
        s0 = fire_scatter(dg, rb0, ssem0, 0)
        for d in g1:
            d.wait()
        s1 = fire_scatter(dg, rb1, ssem1, gg)
        for d in s0:
            d.wait()
        for d in s1:
            d.wait()
        return 0

    lax.fori_loop(0, nsup, sup, 0)

    def tail(t, _):
        b = start + nsup * 2 * gg + t
        pltpu.sync_copy(src2_h.at[pl.ds(b, 1)], sg.at[pl.ds(0, 1)])
        pltpu.sync_copy(dst2_h.at[pl.ds(b, 1)], dg.at[pl.ds(0, 1)])
        pltpu.sync_copy(tab_h.at[sg.at[0]], rb0.at[pl.ds(0, BLK)])
        pltpu.sync_copy(rb0.at[pl.ds(0, BLK)], acc_sp.at[dg.at[0]], add=True)
        return 0

    lax.fori_loop(0, cnt - nsup * 2 * gg, tail, 0)


def _seg128(t8, src, dst):
    @functools.partial(
        pl.kernel,
        out_type=jax.ShapeDtypeStruct((NCH, N, CCH), jnp.float32),
        mesh=_MESH,
        compiler_params=pltpu.CompilerParams(use_tc_tiling_on_sc=False),
        scratch_types=[
            pltpu.VMEM_SHARED((N, CCH), jnp.float32),
            pltpu.VMEM((ZR, CCH), jnp.float32),
            pltpu.VMEM((2 * GG, BLK), jnp.int32),
            pltpu.VMEM((2 * GG, BLK), jnp.int32),
            pltpu.VMEM((GG * BLK, CCH), jnp.float32),
            pltpu.VMEM((GG * BLK, CCH), jnp.float32),
            pltpu.SemaphoreType.DMA,
            pltpu.SemaphoreType.DMA,
            pltpu.SemaphoreType.DMA,
            pltpu.SemaphoreType.DMA,
        ],
    )
    def k(t8_h, src_h, dst_h, out_h, acc_sp, zbuf, sg, dg, rb0, rb1, gsem0,
          gsem1, ssem0, ssem1):
        cid = lax.axis_index("c")
        tid = lax.axis_index("s")
        _fill(zbuf, ZR, CCH, 0.0)
        start, cnt = _span(NBLK, NSUB, tid)

        def chunk(j, _):
            ci = (NCH // NCORE) * cid + j
            _zero_my_rows(acc_sp, zbuf, tid)
            plsc.subcore_barrier()
            _accumulate(GG, t8_h.at[ci], src_h, dst_h, acc_sp, sg, dg, rb0,
                        rb1, gsem0, gsem1, ssem0, ssem1, start, cnt)
            plsc.subcore_barrier()
            _copy_my_rows(acc_sp, out_h.at[ci], tid)
            plsc.subcore_barrier()
            return 0

        lax.fori_loop(0, NCH // NCORE, chunk, 0)

    return k(t8, src, dst)


def _seg16(z0, z1, src0, dst0, src1, dst1):
    @functools.partial(
        pl.kernel,
        out_type=jax.ShapeDtypeStruct((4, N, OUT), jnp.float32),
        mesh=_MESH,
        compiler_params=pltpu.CompilerParams(use_tc_tiling_on_sc=False),
        scratch_types=[
            pltpu.VMEM_SHARED((N, OUT), jnp.float32),
            pltpu.VMEM((ZR, OUT), jnp.float32),
            pltpu.VMEM((2 * GG16, BLK), jnp.int32),
            pltpu.VMEM((2 * GG16, BLK), jnp.int32),
            pltpu.VMEM((GG16 * BLK, OUT), jnp.float32),
            pltpu.VMEM((GG16 * BLK, OUT), jnp.float32),
            pltpu.SemaphoreType.DMA,
            pltpu.SemaphoreType.DMA,
            pltpu.SemaphoreType.DMA,
            pltpu.SemaphoreType.DMA,
        ],
    )
    def k(z0_h, z1_h, s0_h, d0_h, s1_h, d1_h, out_h, acc_sp, zbuf, sg, dg,
          rb0, rb1, gsem0, gsem1, ssem0, ssem1):
        cid = lax.axis_index("c")
        tid = lax.axis_index("s")
        _fill(zbuf, ZR, OUT, 0.0)
        cstart, ccnt = _span(NBLK, NCORE, cid)
        tstart, tcnt = _span(ccnt, NSUB, tid)
        start = cstart + tstart

        for rel, (z_h, s_h, d_h) in enumerate(((z0_h, s0_h, d0_h),
                                               (z1_h, s1_h, d1_h))):
            _zero_my_rows(acc_sp, zbuf, tid)
            plsc.subcore_barrier()
            _accumulate(GG16, z_h, s_h, d_h, acc_sp, sg, dg, rb0, rb1,
                        gsem0, gsem1, ssem0, ssem1, start, tcnt)
            plsc.subcore_barrier()

            @pl.when(cid == 0)
            def _():
                _copy_my_rows(acc_sp, out_h.at[2 * rel], tid)

            @pl.when(cid == 1)
            def _():
                _copy_my_rows(acc_sp, out_h.at[2 * rel + 1], tid)

            plsc.subcore_barrier()

    return k(z0, z1, src0, dst0, src1, dst1)


def _counts(dst_c, dst_cb, dst_f):
    @functools.partial(
        pl.kernel,
        out_type=(jax.ShapeDtypeStruct((N, OUT), jnp.float32),) * 3,
        mesh=_MESH,
        compiler_params=pltpu.CompilerParams(use_tc_tiling_on_sc=False),
        scratch_types=[
            pltpu.VMEM_SHARED((N, OUT), jnp.float32),
            pltpu.VMEM_SHARED((N, OUT), jnp.float32),
            pltpu.VMEM((ZR, OUT), jnp.float32),
            pltpu.VMEM((BLK, OUT), jnp.float32),
            pltpu.VMEM((2 * GG16, BLK), jnp.int32),
            pltpu.VMEM((ZR, OUT), jnp.float32),
            pltpu.SemaphoreType.DMA,
        ],
    )
    def k(dc_h, dcb_h, df_h, qc_h, qcb_h, qf_h, acc0, acc1, zbuf, ones, dbuf,
          qbuf, ssem):
        cid = lax.axis_index("c")
        tid = lax.axis_index("s")
        _fill(zbuf, ZR, OUT, 0.0)
        _fill(ones, BLK, OUT, 1.0)
        start, cnt = _span(NBLK, NSUB, tid)

        def hist(d_h, acc):
            def group(base, nb):
                pltpu.sync_copy(d_h.at[pl.ds(base, nb)],
                                dbuf.at[pl.ds(0, nb)])
                sds = [pltpu.async_copy(ones, acc.at[dbuf.at[j]], ssem,
                                        add=True) for j in range(nb)]
                for d in sds:
                    d.wait()

            ngrp = cnt // (2 * GG16)

            def body(g, _):
                group(start + g * 2 * GG16, 2 * GG16)
                return 0

            lax.fori_loop(0, ngrp, body, 0)

            def tailb(t, _):
                group(start + ngrp * 2 * GG16 + t, 1)
                return 0

            lax.fori_loop(0, cnt - ngrp * 2 * GG16, tailb, 0)

        def finalize(acc, q_h):
            def fn(row0, nch):
                for z in range(nch):
                    r0 = row0 + z * ZR
                    pltpu.sync_copy(acc.at[pl.ds(r0, ZR)], qbuf)

                    def row(i, _):
                        v = qbuf[i, pl.ds(0, 16)]
                        r = 1.0 / jnp.maximum(v, 1.0)
                        qbuf[i, pl.ds(0, 16)] = jnp.where(
                            v > 0.5, r, jnp.zeros((16,), jnp.float32))
                        return 0

                    lax.fori_loop(0, ZR, row, 0)
                    pltpu.sync_copy(qbuf, q_h.at[pl.ds(r0, ZR)])

            _per_tile_rows(tid, fn)

        @pl.when(cid == 0)
        def _():
            _zero_my_rows(acc0, zbuf, tid)
            _zero_my_rows(acc1, zbuf, tid)
            plsc.subcore_barrier()
            hist(dc_h, acc0)
            hist(dcb_h, acc1)
            plsc.subcore_barrier()
            finalize(acc0, qc_h)
            finalize(acc1, qcb_h)

        @pl.when(cid == 1)
        def _():
            _zero_my_rows(acc0, zbuf, tid)
            plsc.subcore_barrier()
            hist(df_h, acc0)
            plsc.subcore_barrier()
            finalize(acc0, qf_h)

    return k(dst_c, dst_cb, dst_f)


RB = 1000
_GRID = N // RB


def _lrelu(x):
    return jnp.where(x >= 0, x, 0.01 * x)


def _chunk_mm(s, w):
    acc = jnp.dot(s[0], w[0:CCH, :], preferred_element_type=jnp.float32)
    for c in range(1, NCH):
        acc = acc + jnp.dot(s[c], w[c * CCH:(c + 1) * CCH, :],
                            preferred_element_type=jnp.float32)
    return acc


def _stage_b_body(sc_r, qc_r, scb_r, qcb_r, sf_r, qf_r, w0c_r, b0c_r, w0cb_r,
                  b0cb_r, w0f_r, b0f_r, w1cb_r, w1f_r, wlin_r, zi_r, zu_r):
    wlin = wlin_r[...]
    qc = qc_r[...][:, 0:1]
    gc = (qc > 0).astype(jnp.float32)
    item0 = qc * _chunk_mm(sc_r[...], w0c_r[...]) + gc * b0c_r[...]
    zi_r[...] = jnp.dot(_lrelu(item0),
                        jnp.dot(w1cb_r[...], wlin,
                                preferred_element_type=jnp.float32),
                        preferred_element_type=jnp.float32)
    qcb = qcb_r[...][:, 0:1]
    gcb = (qcb > 0).astype(jnp.float32)
    qf = qf_r[...][:, 0:1]
    gf = (qf > 0).astype(jnp.float32)
    user0 = (qcb * _chunk_mm(scb_r[...], w0cb_r[...]) + gcb * b0cb_r[...]
             + qf * _chunk_mm(sf_r[...], w0f_r[...]) + gf * b0f_r[...])
    zu_r[...] = jnp.dot(_lrelu(user0),
                        jnp.dot(w1f_r[...], wlin,
                                preferred_element_type=jnp.float32),
                        preferred_element_type=jnp.float32)


def _stage_b(sc, qc, scb, qcb, sf, qf, w0c, b0c, w0cb, b0cb, w0f, b0f, w1cb,
             w1f, wlin):
    s_spec = pl.BlockSpec((NCH, RB, CCH), lambda r: (0, r, 0))
    q_spec = pl.BlockSpec((RB, OUT), lambda r: (r, 0))
    w_spec = pl.BlockSpec((HID, HID), lambda r: (0, 0))
    b_spec = pl.BlockSpec((1, HID), lambda r: (0, 0))
    return pl.pallas_call(
        _stage_b_body,
        grid=(_GRID,),
        in_specs=[s_spec, q_spec, s_spec, q_spec, s_spec, q_spec,
                  w_spec, b_spec, w_spec, b_spec, w_spec, b_spec,
                  w_spec, w_spec, pl.BlockSpec((HID, OUT), lambda r: (0, 0))],
        out_specs=[q_spec, q_spec],
        out_shape=[jax.ShapeDtypeStruct((N, OUT), jnp.float32)] * 2,
    )(sc, qc, scb, qcb, sf, qf, w0c, b0c, w0cb, b0cb, w0f, b0f, w1cb, w1f,
      wlin)


def _stage_d_body(p_r, qcb_r, qf_r, b1cb_r, b1f_r, wlin_r, linb_r, out_r):
    p = p_r[...]
    qcb = qcb_r[...][:, 0:1]
    gcb = (qcb > 0).astype(jnp.float32)
    qf = qf_r[...][:, 0:1]
    gf = (qf > 0).astype(jnp.float32)
    wlin = wlin_r[...]
    bias = (gcb * jnp.dot(b1cb_r[...], wlin, preferred_element_type=jnp.float32)
            + gf * jnp.dot(b1f_r[...], wlin, preferred_element_type=jnp.float32)
            + linb_r[...])
    out_r[...] = qcb * (p[0] + p[1]) + qf * (p[2] + p[3]) + bias


def _stage_d(p, qcb, qf, b1cb, b1f, wlin, linb):
    q_spec = pl.BlockSpec((RB, OUT), lambda r: (r, 0))
    return pl.pallas_call(
        _stage_d_body,
        grid=(_GRID,),
        in_specs=[pl.BlockSpec((4, RB, OUT), lambda r: (0, r, 0)), q_spec,
                  q_spec, pl.BlockSpec((1, HID), lambda r: (0, 0)),
                  pl.BlockSpec((1, HID), lambda r: (0, 0)),
                  pl.BlockSpec((HID, OUT), lambda r: (0, 0)),
                  pl.BlockSpec((1, OUT), lambda r: (0, 0))],
        out_specs=q_spec,
        out_shape=jax.ShapeDtypeStruct((N, OUT), jnp.float32),
    )(p, qcb, qf, b1cb, b1f, wlin, linb)


def kernel(features, embed_item, edge_index_clicks, edge_index_clicked_by,
           edge_index_follows, W0_clicks, b0_clicks, W0_clicked_by,
           b0_clicked_by, W0_follows, b0_follows, W1_clicks, b1_clicks,
           W1_clicked_by, b1_clicked_by, W1_follows, b1_follows, lin_W,
           lin_b):
    i32 = jnp.int32
    r2 = lambda x: x.astype(i32).reshape(NBLK, BLK)
    sc_, dc_ = r2(edge_index_clicks[0]), r2(edge_index_clicks[1])
    scb, dcb = r2(edge_index_clicked_by[0]), r2(edge_index_clicked_by[1])
    sf_, df_ = r2(edge_index_follows[0]), r2(edge_index_follows[1])

    f8 = features.reshape(N, NCH, CCH).transpose(1, 0, 2)
    e8 = embed_item.reshape(N, NCH, CCH).transpose(1, 0, 2)

    qc, qcb, qf = _counts(dc_, dcb, df_)
    s_clicks = _seg128(f8, sc_, dc_)
    s_cb = _seg128(e8, scb, dcb)
    s_f = _seg128(f8, sf_, df_)

    zi, zu = _stage_b(s_clicks, qc, s_cb, qcb, s_f, qf,
                      W0_clicks, b0_clicks.reshape(1, HID),
                      W0_clicked_by, b0_clicked_by.reshape(1, HID),
                      W0_follows, b0_follows.reshape(1, HID),
                      W1_clicked_by, W1_follows, lin_W)

    p = _seg16(zi, zu, scb, dcb, sf_, df_)

    return _stage_d(p, qcb, qf, b1_clicked_by.reshape(1, HID),
                    b1_follows.reshape(1, HID), lin_W,
                    lin_b.reshape(1, OUT))


# speedup vs baseline: 1.1014x; 1.0081x over previous
import functools

import jax
import jax.numpy as jnp
from jax import lax
from jax.experimental import pallas as pl
from jax.experimental.pallas import tpu as pltpu
from jax.experimental.pallas import tpu_sc as plsc

N = 50000
E = 400000
HID = 128
OUT = 16
NSUB = 16
NCORE = 2
BLK = 128
NBLK = E // BLK
CCH = 32
NCH = HID // CCH
ZR = 80
RPT = 3200
GG = 3
GG16 = 8

_MESH = plsc.VectorSubcoreMesh(core_axis_name="c", subcore_axis_name="s")


def _span(total, parts, i):
    base = total // parts
    rem = total % parts
    start = i * base + jnp.minimum(i, rem)
    cnt = base + jnp.where(i < rem, 1, 0).astype(jnp.int32)
    return start, cnt


def _fill(buf, rows, cols, value):
    v = jnp.full((16,), value, jnp.float32)

    def row(i, _):
        for c0 in range(0, cols, 16):
            buf[i, pl.ds(c0, 16)] = v
        return 0

    lax.fori_loop(0, rows, row, 0)


def _per_tile_rows(tid, fn):
    @pl.when(tid < NSUB - 1)
    def _():
        fn(tid * RPT, RPT // ZR)

    @pl.when(tid == NSUB - 1)
    def _():
        fn((NSUB - 1) * RPT, (N - (NSUB - 1) * RPT) // ZR)


def _zero_my_rows(acc_sp, zbuf, tid):
    def fn(row0, nch):
        for z in range(nch):
            pltpu.sync_copy(zbuf, acc_sp.at[pl.ds(row0 + z * ZR, ZR)])

    _per_tile_rows(tid, fn)


def _copy_my_rows(acc_sp, out_ref, tid):
    def fn(row0, nch):
        for z in range(nch):
            pltpu.sync_copy(acc_sp.at[pl.ds(row0 + z * ZR, ZR)],
                            out_ref.at[pl.ds(row0 + z * ZR, ZR)])

    _per_tile_rows(tid, fn)


def _accumulate(gg, tab_h, src2_h, dst2_h, acc_sp, sg, dg, rb0, rb1, gsem0,
                gsem1, ssem0, ssem1, start, cnt):
    def fire_gather(idx, rb, sem, j0):
        return [pltpu.async_copy(tab_h.at[idx.at[j0 + j]],
                                 rb.at[pl.ds(j * BLK, BLK)], sem)
                for j in range(gg)]

    def fire_scatter(idx, rb, sem, j0):
        return [pltpu.async_copy(rb.at[pl.ds(j * BLK, BLK)],
                                 acc_sp.at[idx.at[j0 + j]], sem, add=True)
                for j in range(gg)]

    nsup = cnt // (2 * gg)

    def sup(s, _):
        base = start + s * 2 * gg
        pltpu.sync_copy(src2_h.at[pl.ds(base, 2 * gg)], sg)
        pltpu.sync_copy(dst2_h.at[pl.ds(base, 2 * gg)], dg)
        g0 = fire_gather(sg, rb0, gsem0, 0)
        g1 = fire_gather(sg, rb1, gsem1, gg)
        for d in g0:
            d.wait()
        s0 = fire_scatter(dg, rb0, ssem0, 0)
        for d in g1:
            d.wait()
        s1 = fire_scatter(dg, rb1, ssem1, gg)
        for d in s0:
            d.wait()
        for d in s1:
            d.wait()
        return 0

    lax.fori_loop(0, nsup, sup, 0)

    def tail(t, _):
        b = start + nsup * 2 * gg + t
        pltpu.sync_copy(src2_h.at[pl.ds(b, 1)], sg.at[pl.ds(0, 1)])
        pltpu.sync_copy(dst2_h.at[pl.ds(b, 1)], dg.at[pl.ds(0, 1)])
        pltpu.sync_copy(tab_h.at[sg.at[0]], rb0.at[pl.ds(0, BLK)])
        pltpu.sync_copy(rb0.at[pl.ds(0, BLK)], acc_sp.at[dg.at[0]], add=True)
        return 0

    lax.fori_loop(0, cnt - nsup * 2 * gg, tail, 0)


def _seg128(t4, src, dst):
    @functools.partial(
        pl.kernel,
        out_type=jax.ShapeDtypeStruct((NCH, N, CCH), jnp.float32),
        mesh=_MESH,
        compiler_params=pltpu.CompilerParams(use_tc_tiling_on_sc=False),
        scratch_types=[
            pltpu.VMEM_SHARED((N, CCH), jnp.float32),
            pltpu.VMEM((ZR, CCH), jnp.float32),
            pltpu.VMEM((2 * GG, BLK), jnp.int32),
            pltpu.VMEM((2 * GG, BLK), jnp.int32),
            pltpu.VMEM((GG * BLK, CCH), jnp.float32),
            pltpu.VMEM((GG * BLK, CCH), jnp.float32),
            pltpu.SemaphoreType.DMA,
            pltpu.SemaphoreType.DMA,
            pltpu.SemaphoreType.DMA,
            pltpu.SemaphoreType.DMA,
        ],
    )
    def k(t4_h, src_h, dst_h, out_h, acc_sp, zbuf, sg, dg, rb0, rb1, gsem0,
          gsem1, ssem0, ssem1):
        cid = lax.axis_index("c")
        tid = lax.axis_index("s")
        _fill(zbuf, ZR, CCH, 0.0)
        start, cnt = _span(NBLK, NSUB, tid)

        def do_chunk(ci):
            _zero_my_rows(acc_sp, zbuf, tid)
            plsc.subcore_barrier()
            _accumulate(GG, t4_h.at[ci], src_h, dst_h, acc_sp, sg, dg, rb0,
                        rb1, gsem0, gsem1, ssem0, ssem1, start, cnt)
            plsc.subcore_barrier()
            _copy_my_rows(acc_sp, out_h.at[ci], tid)
            plsc.subcore_barrier()

        for j in range(NCH // NCORE):
            @pl.when(cid == 0)
            def _():
                do_chunk(j)

            @pl.when(cid == 1)
            def _():
                do_chunk(NCH // NCORE + j)

    return k(t4, src, dst)


def _seg16(z0, z1, src0, dst0, src1, dst1):
    @functools.partial(
        pl.kernel,
        out_type=jax.ShapeDtypeStruct((4, N, OUT), jnp.float32),
        mesh=_MESH,
        compiler_params=pltpu.CompilerParams(use_tc_tiling_on_sc=False),
        scratch_types=[
            pltpu.VMEM_SHARED((N, OUT), jnp.float32),
            pltpu.VMEM((ZR, OUT), jnp.float32),
            pltpu.VMEM((2 * GG16, BLK), jnp.int32),
            pltpu.VMEM((2 * GG16, BLK), jnp.int32),
            pltpu.VMEM((GG16 * BLK, OUT), jnp.float32),
            pltpu.VMEM((GG16 * BLK, OUT), jnp.float32),
            pltpu.SemaphoreType.DMA,
            pltpu.SemaphoreType.DMA,
            pltpu.SemaphoreType.DMA,
            pltpu.SemaphoreType.DMA,
        ],
    )
    def k(z0_h, z1_h, s0_h, d0_h, s1_h, d1_h, out_h, acc_sp, zbuf, sg, dg,
          rb0, rb1, gsem0, gsem1, ssem0, ssem1):
        cid = lax.axis_index("c")
        tid = lax.axis_index("s")
        _fill(zbuf, ZR, OUT, 0.0)
        cstart, ccnt = _span(NBLK, NCORE, cid)
        tstart, tcnt = _span(ccnt, NSUB, tid)
        start = cstart + tstart

        for rel, (z_h, s_h, d_h) in enumerate(((z0_h, s0_h, d0_h),
                                               (z1_h, s1_h, d1_h))):
            _zero_my_rows(acc_sp, zbuf, tid)
            plsc.subcore_barrier()
            _accumulate(GG16, z_h, s_h, d_h, acc_sp, sg, dg, rb0, rb1,
                        gsem0, gsem1, ssem0, ssem1, start, tcnt)
            plsc.subcore_barrier()

            @pl.when(cid == 0)
            def _():
                _copy_my_rows(acc_sp, out_h.at[2 * rel], tid)

            @pl.when(cid == 1)
            def _():
                _copy_my_rows(acc_sp, out_h.at[2 * rel + 1], tid)

            plsc.subcore_barrier()

    return k(z0, z1, src0, dst0, src1, dst1)


def _counts(dst_c, dst_cb, dst_f):
    @functools.partial(
        pl.kernel,
        out_type=(jax.ShapeDtypeStruct((N, OUT), jnp.float32),) * 3,
        mesh=_MESH,
        compiler_params=pltpu.CompilerParams(use_tc_tiling_on_sc=False),
        scratch_types=[
            pltpu.VMEM_SHARED((N, OUT), jnp.float32),
            pltpu.VMEM_SHARED((N, OUT), jnp.float32),
            pltpu.VMEM((ZR, OUT), jnp.float32),
            pltpu.VMEM((BLK, OUT), jnp.float32),
            pltpu.VMEM((2 * GG16, BLK), jnp.int32),
            pltpu.SemaphoreType.DMA,
        ],
    )
    def k(dc_h, dcb_h, df_h, qc_h, qcb_h, qf_h, acc0, acc1, zbuf, ones, dbuf,
          ssem):
        cid = lax.axis_index("c")
        tid = lax.axis_index("s")
        _fill(zbuf, ZR, OUT, 0.0)
        _fill(ones, BLK, OUT, 1.0)
        start, cnt = _span(NBLK, NSUB, tid)

        def hist(d_h, acc):
            def group(base, nb):
                pltpu.sync_copy(d_h.at[pl.ds(base, nb)],
                                dbuf.at[pl.ds(0, nb)])
                sds = [pltpu.async_copy(ones, acc.at[dbuf.at[j]], ssem,
                                        add=True) for j in range(nb)]
                for d in sds:
                    d.wait()

            ngrp = cnt // (2 * GG16)

            def body(g, _):
                group(start + g * 2 * GG16, 2 * GG16)
                return 0

            lax.fori_loop(0, ngrp, body, 0)

            def tailb(t, _):
                group(start + ngrp * 2 * GG16 + t, 1)
                return 0

            lax.fori_loop(0, cnt - ngrp * 2 * GG16, tailb, 0)

        @pl.when(cid == 0)
        def _():
            _zero_my_rows(acc0, zbuf, tid)
            _zero_my_rows(acc1, zbuf, tid)
            plsc.subcore_barrier()
            hist(dc_h, acc0)
            hist(dcb_h, acc1)
            plsc.subcore_barrier()
            _copy_my_rows(acc0, qc_h, tid)
            _copy_my_rows(acc1, qcb_h, tid)

        @pl.when(cid == 1)
        def _():
            _zero_my_rows(acc0, zbuf, tid)
            plsc.subcore_barrier()
            hist(df_h, acc0)
            plsc.subcore_barrier()
            _copy_my_rows(acc0, qf_h, tid)

    return k(dst_c, dst_cb, dst_f)


RB = 1000
_GRID = N // RB


def _lrelu(x):
    return jnp.where(x >= 0, x, 0.01 * x)


def _qgate(cnt_r):
    c = cnt_r[...][:, 0:1]
    q = jnp.where(c > 0.5, 1.0 / jnp.maximum(c, 1.0), 0.0)
    return q, (c > 0.5).astype(jnp.float32)


def _chunk_mm(s, w):
    acc = jnp.dot(s[0], w[0:CCH, :], preferred_element_type=jnp.float32)
    for c in range(1, NCH):
        acc = acc + jnp.dot(s[c], w[c * CCH:(c + 1) * CCH, :],
                            preferred_element_type=jnp.float32)
    return acc


def _stage_b_body(sc_r, qc_r, scb_r, qcb_r, sf_r, qf_r, w0c_r, b0c_r, w0cb_r,
                  b0cb_r, w0f_r, b0f_r, w1cb_r, w1f_r, wlin_r, zi_r, zu_r):
    wlin = wlin_r[...]
    qc, gc = _qgate(qc_r)
    item0 = qc * _chunk_mm(sc_r[...], w0c_r[...]) + gc * b0c_r[...]
    zi_r[...] = jnp.dot(_lrelu(item0),
                        jnp.dot(w1cb_r[...], wlin,
                                preferred_element_type=jnp.float32),
                        preferred_element_type=jnp.float32)
    qcb, gcb = _qgate(qcb_r)
    qf, gf = _qgate(qf_r)
    user0 = (qcb * _chunk_mm(scb_r[...], w0cb_r[...]) + gcb * b0cb_r[...]
             + qf * _chunk_mm(sf_r[...], w0f_r[...]) + gf * b0f_r[...])
    zu_r[...] = jnp.dot(_lrelu(user0),
                        jnp.dot(w1f_r[...], wlin,
                                preferred_element_type=jnp.float32),
                        preferred_element_type=jnp.float32)


def _stage_b(sc, qc, scb, qcb, sf, qf, w0c, b0c, w0cb, b0cb, w0f, b0f, w1cb,
             w1f, wlin):
    s_spec = pl.BlockSpec((NCH, RB, CCH), lambda r: (0, r, 0))
    q_spec = pl.BlockSpec((RB, OUT), lambda r: (r, 0))
    w_spec = pl.BlockSpec((HID, HID), lambda r: (0, 0))
    b_spec = pl.BlockSpec((1, HID), lambda r: (0, 0))
    return pl.pallas_call(
        _stage_b_body,
        grid=(_GRID,),
        in_specs=[s_spec, q_spec, s_spec, q_spec, s_spec, q_spec,
                  w_spec, b_spec, w_spec, b_spec, w_spec, b_spec,
                  w_spec, w_spec, pl.BlockSpec((HID, OUT), lambda r: (0, 0))],
        out_specs=[q_spec, q_spec],
        out_shape=[jax.ShapeDtypeStruct((N, OUT), jnp.float32)] * 2,
    )(sc, qc, scb, qcb, sf, qf, w0c, b0c, w0cb, b0cb, w0f, b0f, w1cb, w1f,
      wlin)


def _stage_d_body(p_r, qcb_r, qf_r, b1cb_r, b1f_r, wlin_r, linb_r, out_r):
    p = p_r[...]
    qcb, gcb = _qgate(qcb_r)
    qf, gf = _qgate(qf_r)
    wlin = wlin_r[...]
    bias = (gcb * jnp.dot(b1cb_r[...], wlin, preferred_element_type=jnp.float32)
            + gf * jnp.dot(b1f_r[...], wlin, preferred_element_type=jnp.float32)
            + linb_r[...])
    out_r[...] = qcb * (p[0] + p[1]) + qf * (p[2] + p[3]) + bias


def _stage_d(p, qcb, qf, b1cb, b1f, wlin, linb):
    q_spec = pl.BlockSpec((RB, OUT), lambda r: (r, 0))
    return pl.pallas_call(
        _stage_d_body,
        grid=(_GRID,),
        in_specs=[pl.BlockSpec((4, RB, OUT), lambda r: (0, r, 0)), q_spec,
                  q_spec, pl.BlockSpec((1, HID), lambda r: (0, 0)),
                  pl.BlockSpec((1, HID), lambda r: (0, 0)),
                  pl.BlockSpec((HID, OUT), lambda r: (0, 0)),
                  pl.BlockSpec((1, OUT), lambda r: (0, 0))],
        out_specs=q_spec,
        out_shape=jax.ShapeDtypeStruct((N, OUT), jnp.float32),
    )(p, qcb, qf, b1cb, b1f, wlin, linb)


def kernel(features, embed_item, edge_index_clicks, edge_index_clicked_by,
           edge_index_follows, W0_clicks, b0_clicks, W0_clicked_by,
           b0_clicked_by, W0_follows, b0_follows, W1_clicks, b1_clicks,
           W1_clicked_by, b1_clicked_by, W1_follows, b1_follows, lin_W,
           lin_b):
    i32 = jnp.int32
    r2 = lambda x: x.astype(i32).reshape(NBLK, BLK)
    sc_, dc_ = r2(edge_index_clicks[0]), r2(edge_index_clicks[1])
    scb, dcb = r2(edge_index_clicked_by[0]), r2(edge_index_clicked_by[1])
    sf_, df_ = r2(edge_index_follows[0]), r2(edge_index_follows[1])

    f4 = features.reshape(N, NCH, CCH).transpose(1, 0, 2)
    e4 = embed_item.reshape(N, NCH, CCH).transpose(1, 0, 2)

    qc, qcb, qf = _counts(dc_, dcb, df_)
    s_clicks = _seg128(f4, sc_, dc_)
    s_cb = _seg128(e4, scb, dcb)
    s_f = _seg128(f4, sf_, df_)

    zi, zu = _stage_b(s_clicks, qc, s_cb, qcb, s_f, qf,
                      W0_clicks, b0_clicks.reshape(1, HID),
                      W0_clicked_by, b0_clicked_by.reshape(1, HID),
                      W0_follows, b0_follows.reshape(1, HID),
                      W1_clicked_by, W1_follows, lin_W)

    p = _seg16(zi, zu, scb, dcb, sf_, df_)

    return _stage_d(p, qcb, qf, b1_clicked_by.reshape(1, HID),
                    b1_follows.reshape(1, HID), lin_W,
                    lin_b.reshape(1, OUT))


# speedup vs baseline: 1.1564x; 1.0500x over previous
import functools

import jax
import jax.numpy as jnp
from jax import lax
from jax.experimental import pallas as pl
from jax.experimental.pallas import tpu as pltpu
from jax.experimental.pallas import tpu_sc as plsc

N = 50000
E = 400000
HID = 128
OUT = 16
NSUB = 16
NCORE = 2
BLK = 128
NBLK = E // BLK
CCH = 32
NCH = HID // CCH
ZR = 80
RPT = 3200
GG = 3
GG16 = 8
NU = 4

_MESH = plsc.VectorSubcoreMesh(core_axis_name="c", subcore_axis_name="s")


def _span(total, parts, i):
    base = total // parts
    rem = total % parts
    start = i * base + jnp.minimum(i, rem)
    cnt = base + jnp.where(i < rem, 1, 0).astype(jnp.int32)
    return start, cnt


def _fill(buf, rows, cols, value):
    v = jnp.full((16,), value, jnp.float32)

    def row(i, _):
        for c0 in range(0, cols, 16):
            buf[i, pl.ds(c0, 16)] = v
        return 0

    lax.fori_loop(0, rows, row, 0)


def _per_tile_rows(tid, fn):
    @pl.when(tid < NSUB - 1)
    def _():
        fn(tid * RPT, RPT // ZR)

    @pl.when(tid == NSUB - 1)
    def _():
        fn((NSUB - 1) * RPT, (N - (NSUB - 1) * RPT) // ZR)


def _zero_my_rows(acc_sp, zbuf, tid):
    def fn(row0, nch):
        for z in range(nch):
            pltpu.sync_copy(zbuf, acc_sp.at[pl.ds(row0 + z * ZR, ZR)])

    _per_tile_rows(tid, fn)


def _copy_my_rows(acc_sp, out_ref, tid):
    def fn(row0, nch):
        for z in range(nch):
            pltpu.sync_copy(acc_sp.at[pl.ds(row0 + z * ZR, ZR)],
                            out_ref.at[pl.ds(row0 + z * ZR, ZR)])

    _per_tile_rows(tid, fn)


def _accumulate(gg, tab_h, src2_h, dst2_h, acc_sp, sg, dg, rb0, rb1, gsem0,
                gsem1, ssem0, ssem1, start, cnt):
    rbs = (rb0, rb1)
    gsems = (gsem0, gsem1)
    ssems = (ssem0, ssem1)

    def fire_gather(u):
        rb = rbs[u % 2]
        return [pltpu.async_copy(tab_h.at[sg.at[u * gg + j]],
                                 rb.at[pl.ds(j * BLK, BLK)], gsems[u % 2])
                for j in range(gg)]

    def fire_scatter(u):
        rb = rbs[u % 2]
        return [pltpu.async_copy(rb.at[pl.ds(j * BLK, BLK)],
                                 acc_sp.at[dg.at[u * gg + j]], ssems[u % 2],
                                 add=True)
                for j in range(gg)]

    nsup = cnt // (NU * gg)

    def sup(s, _):
        base = start + s * NU * gg
        pltpu.sync_copy(src2_h.at[pl.ds(base, NU * gg)], sg)
        pltpu.sync_copy(dst2_h.at[pl.ds(base, NU * gg)], dg)
        g = [fire_gather(0), fire_gather(1)] + [None] * (NU - 2)
        sc = [None] * NU
        for u in range(NU):
            for d in g[u]:
                d.wait()
            sc[u] = fire_scatter(u)
            if u >= 1 and u + 1 < NU:
                for d in sc[u - 1]:
                    d.wait()
                g[u + 1] = fire_gather(u + 1)
        for d in sc[NU - 2]:
            d.wait()
        for d in sc[NU - 1]:
            d.wait()
        return 0

    lax.fori_loop(0, nsup, sup, 0)

    def tail(t, _):
        b = start + nsup * NU * gg + t
        pltpu.sync_copy(src2_h.at[pl.ds(b, 1)], sg.at[pl.ds(0, 1)])
        pltpu.sync_copy(dst2_h.at[pl.ds(b, 1)], dg.at[pl.ds(0, 1)])
        pltpu.sync_copy(tab_h.at[sg.at[0]], rb0.at[pl.ds(0, BLK)])
        pltpu.sync_copy(rb0.at[pl.ds(0, BLK)], acc_sp.at[dg.at[0]], add=True)
        return 0

    lax.fori_loop(0, cnt - nsup * NU * gg, tail, 0)


def _seg128(t4, src, dst):
    @functools.partial(
        pl.kernel,
        out_type=jax.ShapeDtypeStruct((NCH, N, CCH), jnp.float32),
        mesh=_MESH,
        compiler_params=pltpu.CompilerParams(use_tc_tiling_on_sc=False),
        scratch_types=[
            pltpu.VMEM_SHARED((N, CCH), jnp.float32),
            pltpu.VMEM((ZR, CCH), jnp.float32),
            pltpu.VMEM((NU * GG, BLK), jnp.int32),
            pltpu.VMEM((NU * GG, BLK), jnp.int32),
            pltpu.VMEM((GG * BLK, CCH), jnp.float32),
            pltpu.VMEM((GG * BLK, CCH), jnp.float32),
            pltpu.SemaphoreType.DMA,
            pltpu.SemaphoreType.DMA,
            pltpu.SemaphoreType.DMA,
            pltpu.SemaphoreType.DMA,
        ],
    )
    def k(t4_h, src_h, dst_h, out_h, acc_sp, zbuf, sg, dg, rb0, rb1, gsem0,
          gsem1, ssem0, ssem1):
        cid = lax.axis_index("c")
        tid = lax.axis_index("s")
        _fill(zbuf, ZR, CCH, 0.0)
        start, cnt = _span(NBLK, NSUB, tid)

        def do_chunk(ci):
            _zero_my_rows(acc_sp, zbuf, tid)
            plsc.subcore_barrier()
            _accumulate(GG, t4_h.at[ci], src_h, dst_h, acc_sp, sg, dg, rb0,
                        rb1, gsem0, gsem1, ssem0, ssem1, start, cnt)
            plsc.subcore_barrier()
            _copy_my_rows(acc_sp, out_h.at[ci], tid)
            plsc.subcore_barrier()

        for j in range(NCH // NCORE):
            @pl.when(cid == 0)
            def _():
                do_chunk(j)

            @pl.when(cid == 1)
            def _():
                do_chunk(NCH // NCORE + j)

    return k(t4, src, dst)


def _seg16(z0, z1, src0, dst0, src1, dst1):
    @functools.partial(
        pl.kernel,
        out_type=jax.ShapeDtypeStruct((4, N, OUT), jnp.float32),
        mesh=_MESH,
        compiler_params=pltpu.CompilerParams(use_tc_tiling_on_sc=False),
        scratch_types=[
            pltpu.VMEM_SHARED((N, OUT), jnp.float32),
            pltpu.VMEM((ZR, OUT), jnp.float32),
            pltpu.VMEM((NU * GG16, BLK), jnp.int32),
            pltpu.VMEM((NU * GG16, BLK), jnp.int32),
            pltpu.VMEM((GG16 * BLK, OUT), jnp.float32),
            pltpu.VMEM((GG16 * BLK, OUT), jnp.float32),
            pltpu.SemaphoreType.DMA,
            pltpu.SemaphoreType.DMA,
            pltpu.SemaphoreType.DMA,
            pltpu.SemaphoreType.DMA,
        ],
    )
    def k(z0_h, z1_h, s0_h, d0_h, s1_h, d1_h, out_h, acc_sp, zbuf, sg, dg,
          rb0, rb1, gsem0, gsem1, ssem0, ssem1):
        cid = lax.axis_index("c")
        tid = lax.axis_index("s")
        _fill(zbuf, ZR, OUT, 0.0)
        cstart, ccnt = _span(NBLK, NCORE, cid)
        tstart, tcnt = _span(ccnt, NSUB, tid)
        start = cstart + tstart

        for rel, (z_h, s_h, d_h) in enumerate(((z0_h, s0_h, d0_h),
                                               (z1_h, s1_h, d1_h))):
            _zero_my_rows(acc_sp, zbuf, tid)
            plsc.subcore_barrier()
            _accumulate(GG16, z_h, s_h, d_h, acc_sp, sg, dg, rb0, rb1,
                        gsem0, gsem1, ssem0, ssem1, start, tcnt)
            plsc.subcore_barrier()

            @pl.when(cid == 0)
            def _():
                _copy_my_rows(acc_sp, out_h.at[2 * rel], tid)

            @pl.when(cid == 1)
            def _():
                _copy_my_rows(acc_sp, out_h.at[2 * rel + 1], tid)

            plsc.subcore_barrier()

    return k(z0, z1, src0, dst0, src1, dst1)


def _counts(dst_c, dst_cb, dst_f):
    @functools.partial(
        pl.kernel,
        out_type=(jax.ShapeDtypeStruct((N, OUT), jnp.float32),) * 3,
        mesh=_MESH,
        compiler_params=pltpu.CompilerParams(use_tc_tiling_on_sc=False),
        scratch_types=[
            pltpu.VMEM_SHARED((N, OUT), jnp.float32),
            pltpu.VMEM_SHARED((N, OUT), jnp.float32),
            pltpu.VMEM((ZR, OUT), jnp.float32),
            pltpu.VMEM((BLK, OUT), jnp.float32),
            pltpu.VMEM((2 * GG16, BLK), jnp.int32),
            pltpu.SemaphoreType.DMA,
        ],
    )
    def k(dc_h, dcb_h, df_h, qc_h, qcb_h, qf_h, acc0, acc1, zbuf, ones, dbuf,
          ssem):
        cid = lax.axis_index("c")
        tid = lax.axis_index("s")
        _fill(zbuf, ZR, OUT, 0.0)
        _fill(ones, BLK, OUT, 1.0)
        start, cnt = _span(NBLK, NSUB, tid)

        def hist(d_h, acc):
            def group(base, nb):
                pltpu.sync_copy(d_h.at[pl.ds(base, nb)],
                                dbuf.at[pl.ds(0, nb)])
                sds = [pltpu.async_copy(ones, acc.at[dbuf.at[j]], ssem,
                                        add=True) for j in range(nb)]
                for d in sds:
                    d.wait()

            ngrp = cnt // (2 * GG16)

            def body(g, _):
                group(start + g * 2 * GG16, 2 * GG16)
                return 0

            lax.fori_loop(0, ngrp, body, 0)

            def tailb(t, _):
                group(start + ngrp * 2 * GG16 + t, 1)
                return 0

            lax.fori_loop(0, cnt - ngrp * 2 * GG16, tailb, 0)

        @pl.when(cid == 0)
        def _():
            _zero_my_rows(acc0, zbuf, tid)
            _zero_my_rows(acc1, zbuf, tid)
            plsc.subcore_barrier()
            hist(dc_h, acc0)
            hist(dcb_h, acc1)
            plsc.subcore_barrier()
            _copy_my_rows(acc0, qc_h, tid)
            _copy_my_rows(acc1, qcb_h, tid)

        @pl.when(cid == 1)
        def _():
            _zero_my_rows(acc0, zbuf, tid)
            plsc.subcore_barrier()
            hist(df_h, acc0)
            plsc.subcore_barrier()
            _copy_my_rows(acc0, qf_h, tid)

    return k(dst_c, dst_cb, dst_f)


RB = 1000
_GRID = N // RB


def _lrelu(x):
    return jnp.where(x >= 0, x, 0.01 * x)


def _qgate(cnt_r):
    c = cnt_r[...][:, 0:1]
    q = jnp.where(c > 0.5, 1.0 / jnp.maximum(c, 1.0), 0.0)
    return q, (c > 0.5).astype(jnp.float32)


def _chunk_mm(s, w):
    acc = jnp.dot(s[0], w[0:CCH, :], preferred_element_type=jnp.float32)
    for c in range(1, NCH):
        acc = acc + jnp.dot(s[c], w[c * CCH:(c + 1) * CCH, :],
                            preferred_element_type=jnp.float32)
    return acc


def _stage_b_body(sc_r, qc_r, scb_r, qcb_r, sf_r, qf_r, w0c_r, b0c_r, w0cb_r,
                  b0cb_r, w0f_r, b0f_r, w1cb_r, w1f_r, wlin_r, zi_r, zu_r):
    wlin = wlin_r[...]
    qc, gc = _qgate(qc_r)
    item0 = qc * _chunk_mm(sc_r[...], w0c_r[...]) + gc * b0c_r[...]
    zi_r[...] = jnp.dot(_lrelu(item0),
                        jnp.dot(w1cb_r[...], wlin,
                                preferred_element_type=jnp.float32),
                        preferred_element_type=jnp.float32)
    qcb, gcb = _qgate(qcb_r)
    qf, gf = _qgate(qf_r)
    user0 = (qcb * _chunk_mm(scb_r[...], w0cb_r[...]) + gcb * b0cb_r[...]
             + qf * _chunk_mm(sf_r[...], w0f_r[...]) + gf * b0f_r[...])
    zu_r[...] = jnp.dot(_lrelu(user0),
                        jnp.dot(w1f_r[...], wlin,
                                preferred_element_type=jnp.float32),
                        preferred_element_type=jnp.float32)


def _stage_b(sc, qc, scb, qcb, sf, qf, w0c, b0c, w0cb, b0cb, w0f, b0f, w1cb,
             w1f, wlin):
    s_spec = pl.BlockSpec((NCH, RB, CCH), lambda r: (0, r, 0))
    q_spec = pl.BlockSpec((RB, OUT), lambda r: (r, 0))
    w_spec = pl.BlockSpec((HID, HID), lambda r: (0, 0))
    b_spec = pl.BlockSpec((1, HID), lambda r: (0, 0))
    return pl.pallas_call(
        _stage_b_body,
        grid=(_GRID,),
        in_specs=[s_spec, q_spec, s_spec, q_spec, s_spec, q_spec,
                  w_spec, b_spec, w_spec, b_spec, w_spec, b_spec,
                  w_spec, w_spec, pl.BlockSpec((HID, OUT), lambda r: (0, 0))],
        out_specs=[q_spec, q_spec],
        out_shape=[jax.ShapeDtypeStruct((N, OUT), jnp.float32)] * 2,
    )(sc, qc, scb, qcb, sf, qf, w0c, b0c, w0cb, b0cb, w0f, b0f, w1cb, w1f,
      wlin)


def _stage_d_body(p_r, qcb_r, qf_r, b1cb_r, b1f_r, wlin_r, linb_r, out_r):
    p = p_r[...]
    qcb, gcb = _qgate(qcb_r)
    qf, gf = _qgate(qf_r)
    wlin = wlin_r[...]
    bias = (gcb * jnp.dot(b1cb_r[...], wlin, preferred_element_type=jnp.float32)
            + gf * jnp.dot(b1f_r[...], wlin, preferred_element_type=jnp.float32)
            + linb_r[...])
    out_r[...] = qcb * (p[0] + p[1]) + qf * (p[2] + p[3]) + bias


def _stage_d(p, qcb, qf, b1cb, b1f, wlin, linb):
    q_spec = pl.BlockSpec((RB, OUT), lambda r: (r, 0))
    return pl.pallas_call(
        _stage_d_body,
        grid=(_GRID,),
        in_specs=[pl.BlockSpec((4, RB, OUT), lambda r: (0, r, 0)), q_spec,
                  q_spec, pl.BlockSpec((1, HID), lambda r: (0, 0)),
                  pl.BlockSpec((1, HID), lambda r: (0, 0)),
                  pl.BlockSpec((HID, OUT), lambda r: (0, 0)),
                  pl.BlockSpec((1, OUT), lambda r: (0, 0))],
        out_specs=q_spec,
        out_shape=jax.ShapeDtypeStruct((N, OUT), jnp.float32),
    )(p, qcb, qf, b1cb, b1f, wlin, linb)


def kernel(features, embed_item, edge_index_clicks, edge_index_clicked_by,
           edge_index_follows, W0_clicks, b0_clicks, W0_clicked_by,
           b0_clicked_by, W0_follows, b0_follows, W1_clicks, b1_clicks,
           W1_clicked_by, b1_clicked_by, W1_follows, b1_follows, lin_W,
           lin_b):
    i32 = jnp.int32
    r2 = lambda x: x.astype(i32).reshape(NBLK, BLK)
    sc_, dc_ = r2(edge_index_clicks[0]), r2(edge_index_clicks[1])
    scb, dcb = r2(edge_index_clicked_by[0]), r2(edge_index_clicked_by[1])
    sf_, df_ = r2(edge_index_follows[0]), r2(edge_index_follows[1])

    f4 = features.reshape(N, NCH, CCH).transpose(1, 0, 2)
    e4 = embed_item.reshape(N, NCH, CCH).transpose(1, 0, 2)

    qc, qcb, qf = _counts(dc_, dcb, df_)
    s_clicks = _seg128(f4, sc_, dc_)
    s_cb = _seg128(e4, scb, dcb)
    s_f = _seg128(f4, sf_, df_)

    zi, zu = _stage_b(s_clicks, qc, s_cb, qcb, s_f, qf,
                      W0_clicks, b0_clicks.reshape(1, HID),
                      W0_clicked_by, b0_clicked_by.reshape(1, HID),
                      W0_follows, b0_follows.reshape(1, HID),
                      W1_clicked_by, W1_follows, lin_W)

    p = _seg16(zi, zu, scb, dcb, sf_, df_)

    return _stage_d(p, qcb, qf, b1_clicked_by.reshape(1, HID),
                    b1_follows.reshape(1, HID), lin_W,
                    lin_b.reshape(1, OUT))


# speedup vs baseline: 1.1827x; 1.0227x over previous
import functools

import jax
import jax.numpy as jnp
from jax import lax
from jax.experimental import pallas as pl
from jax.experimental.pallas import tpu as pltpu
from jax.experimental.pallas import tpu_sc as plsc

N = 50000
E = 400000
HID = 128
OUT = 16
NSUB = 16
NCORE = 2
BLK = 128
NBLK = E // BLK
CCH = 32
NCH = HID // CCH
ZR = 80
RPT = 3200
GG = 3
GG16 = 8
NU = 4

_MESH = plsc.VectorSubcoreMesh(core_axis_name="c", subcore_axis_name="s")


def _span(total, parts, i):
    base = total // parts
    rem = total % parts
    start = i * base + jnp.minimum(i, rem)
    cnt = base + jnp.where(i < rem, 1, 0).astype(jnp.int32)
    return start, cnt


def _fill(buf, rows, cols, value):
    v = jnp.full((16,), value, jnp.float32)

    def row(i, _):
        for c0 in range(0, cols, 16):
            buf[i, pl.ds(c0, 16)] = v
        return 0

    lax.fori_loop(0, rows, row, 0)


def _per_tile_rows(tid, fn):
    @pl.when(tid < NSUB - 1)
    def _():
        fn(tid * RPT, RPT // ZR)

    @pl.when(tid == NSUB - 1)
    def _():
        fn((NSUB - 1) * RPT, (N - (NSUB - 1) * RPT) // ZR)


def _zero_my_rows(acc_sp, zbuf, tid):
    def fn(row0, nch):
        for z in range(nch):
            pltpu.sync_copy(zbuf, acc_sp.at[pl.ds(row0 + z * ZR, ZR)])

    _per_tile_rows(tid, fn)


def _copy_my_rows(acc_sp, out_ref, tid):
    def fn(row0, nch):
        for z in range(nch):
            pltpu.sync_copy(acc_sp.at[pl.ds(row0 + z * ZR, ZR)],
                            out_ref.at[pl.ds(row0 + z * ZR, ZR)])

    _per_tile_rows(tid, fn)


def _accumulate(gg, tab_h, src2_h, dst2_h, acc_sp, sg, dg, rb0, rb1, gsem0,
                gsem1, ssem0, ssem1, start, cnt):
    rbs = (rb0, rb1)
    gsems = (gsem0, gsem1)
    ssems = (ssem0, ssem1)

    def fire_gather(u):
        rb = rbs[u % 2]
        return [pltpu.async_copy(tab_h.at[sg.at[u * gg + j]],
                                 rb.at[pl.ds(j * BLK, BLK)], gsems[u % 2])
                for j in range(gg)]

    def fire_scatter(u):
        rb = rbs[u % 2]
        return [pltpu.async_copy(rb.at[pl.ds(j * BLK, BLK)],
                                 acc_sp.at[dg.at[u * gg + j]], ssems[u % 2],
                                 add=True)
                for j in range(gg)]

    nsup = cnt // (NU * gg)

    def sup(s, _):
        base = start + s * NU * gg
        pltpu.sync_copy(src2_h.at[pl.ds(base, NU * gg)], sg)
        pltpu.sync_copy(dst2_h.at[pl.ds(base, NU * gg)], dg)
        g = [fire_gather(0), fire_gather(1)] + [None] * (NU - 2)
        sc = [None] * NU
        for u in range(NU):
            for d in g[u]:
                d.wait()
            sc[u] = fire_scatter(u)
            if u >= 1 and u + 1 < NU:
                for d in sc[u - 1]:
                    d.wait()
                g[u + 1] = fire_gather(u + 1)
        for d in sc[NU - 2]:
            d.wait()
        for d in sc[NU - 1]:
            d.wait()
        return 0

    lax.fori_loop(0, nsup, sup, 0)

    def tail(t, _):
        b = start + nsup * NU * gg + t
        pltpu.sync_copy(src2_h.at[pl.ds(b, 1)], sg.at[pl.ds(0, 1)])
        pltpu.sync_copy(dst2_h.at[pl.ds(b, 1)], dg.at[pl.ds(0, 1)])
        pltpu.sync_copy(tab_h.at[sg.at[0]], rb0.at[pl.ds(0, BLK)])
        pltpu.sync_copy(rb0.at[pl.ds(0, BLK)], acc_sp.at[dg.at[0]], add=True)
        return 0

    lax.fori_loop(0, cnt - nsup * NU * gg, tail, 0)


def _seg128(t4, src, dst):
    @functools.partial(
        pl.kernel,
        out_type=jax.ShapeDtypeStruct((NCH, N, CCH), jnp.float32),
        mesh=_MESH,
        compiler_params=pltpu.CompilerParams(use_tc_tiling_on_sc=False),
        scratch_types=[
            pltpu.VMEM_SHARED((N, CCH), jnp.float32),
            pltpu.VMEM((ZR, CCH), jnp.float32),
            pltpu.VMEM((NU * GG, BLK), jnp.int32),
            pltpu.VMEM((NU * GG, BLK), jnp.int32),
            pltpu.VMEM((GG * BLK, CCH), jnp.float32),
            pltpu.VMEM((GG * BLK, CCH), jnp.float32),
            pltpu.SemaphoreType.DMA,
            pltpu.SemaphoreType.DMA,
            pltpu.SemaphoreType.DMA,
            pltpu.SemaphoreType.DMA,
        ],
    )
    def k(t4_h, src_h, dst_h, out_h, acc_sp, zbuf, sg, dg, rb0, rb1, gsem0,
          gsem1, ssem0, ssem1):
        cid = lax.axis_index("c")
        tid = lax.axis_index("s")
        _fill(zbuf, ZR, CCH, 0.0)
        start, cnt = _span(NBLK, NSUB, tid)

        def do_chunk(ci):
            _zero_my_rows(acc_sp, zbuf, tid)
            plsc.subcore_barrier()
            _accumulate(GG, t4_h.at[ci], src_h, dst_h, acc_sp, sg, dg, rb0,
                        rb1, gsem0, gsem1, ssem0, ssem1, start, cnt)
            plsc.subcore_barrier()
            _copy_my_rows(acc_sp, out_h.at[ci], tid)
            plsc.subcore_barrier()

        for j in range(NCH // NCORE):
            @pl.when(cid == 0)
            def _():
                do_chunk(j)

            @pl.when(cid == 1)
            def _():
                do_chunk(NCH // NCORE + j)

    return k(t4, src, dst)


def _seg16(z0, z1, src0, dst0, src1, dst1):
    @functools.partial(
        pl.kernel,
        out_type=(jax.ShapeDtypeStruct((N, OUT), jnp.float32),) * 4,
        mesh=_MESH,
        compiler_params=pltpu.CompilerParams(use_tc_tiling_on_sc=False),
        scratch_types=[
            pltpu.VMEM_SHARED((N, OUT), jnp.float32),
            pltpu.VMEM((ZR, OUT), jnp.float32),
            pltpu.VMEM((NU * GG16, BLK), jnp.int32),
            pltpu.VMEM((NU * GG16, BLK), jnp.int32),
            pltpu.VMEM((GG16 * BLK, OUT), jnp.float32),
            pltpu.VMEM((GG16 * BLK, OUT), jnp.float32),
            pltpu.SemaphoreType.DMA,
            pltpu.SemaphoreType.DMA,
            pltpu.SemaphoreType.DMA,
            pltpu.SemaphoreType.DMA,
        ],
    )
    def k(z0_h, z1_h, s0_h, d0_h, s1_h, d1_h, p0_h, p1_h, p2_h, p3_h,
          acc_sp, zbuf, sg, dg, rb0, rb1, gsem0, gsem1, ssem0, ssem1):
        cid = lax.axis_index("c")
        tid = lax.axis_index("s")
        _fill(zbuf, ZR, OUT, 0.0)
        cstart, ccnt = _span(NBLK, NCORE, cid)
        tstart, tcnt = _span(ccnt, NSUB, tid)
        start = cstart + tstart

        for rel, (z_h, s_h, d_h) in enumerate(((z0_h, s0_h, d0_h),
                                               (z1_h, s1_h, d1_h))):
            _zero_my_rows(acc_sp, zbuf, tid)
            plsc.subcore_barrier()
            _accumulate(GG16, z_h, s_h, d_h, acc_sp, sg, dg, rb0, rb1,
                        gsem0, gsem1, ssem0, ssem1, start, tcnt)
            plsc.subcore_barrier()

            outs = ((p0_h, p1_h), (p2_h, p3_h))[rel]

            @pl.when(cid == 0)
            def _():
                _copy_my_rows(acc_sp, outs[0], tid)

            @pl.when(cid == 1)
            def _():
                _copy_my_rows(acc_sp, outs[1], tid)

            plsc.subcore_barrier()

    return k(z0, z1, src0, dst0, src1, dst1)


def _counts(dst_c, dst_cb, dst_f):
    @functools.partial(
        pl.kernel,
        out_type=(jax.ShapeDtypeStruct((N, OUT), jnp.float32),) * 3,
        mesh=_MESH,
        compiler_params=pltpu.CompilerParams(use_tc_tiling_on_sc=False),
        scratch_types=[
            pltpu.VMEM_SHARED((N, OUT), jnp.float32),
            pltpu.VMEM_SHARED((N, OUT), jnp.float32),
            pltpu.VMEM((ZR, OUT), jnp.float32),
            pltpu.VMEM((BLK, OUT), jnp.float32),
            pltpu.VMEM((2 * GG16, BLK), jnp.int32),
            pltpu.SemaphoreType.DMA,
        ],
    )
    def k(dc_h, dcb_h, df_h, qc_h, qcb_h, qf_h, acc0, acc1, zbuf, ones, dbuf,
          ssem):
        cid = lax.axis_index("c")
        tid = lax.axis_index("s")
        _fill(zbuf, ZR, OUT, 0.0)
        _fill(ones, BLK, OUT, 1.0)
        start, cnt = _span(NBLK, NSUB, tid)

        def hist(d_h, acc):
            def group(base, nb):
                pltpu.sync_copy(d_h.at[pl.ds(base, nb)],
                                dbuf.at[pl.ds(0, nb)])
                sds = [pltpu.async_copy(ones, acc.at[dbuf.at[j]], ssem,
                                        add=True) for j in range(nb)]
                for d in sds:
                    d.wait()

            ngrp = cnt // (2 * GG16)

            def body(g, _):
                group(start + g * 2 * GG16, 2 * GG16)
                return 0

            lax.fori_loop(0, ngrp, body, 0)

            def tailb(t, _):
                group(start + ngrp * 2 * GG16 + t, 1)
                return 0

            lax.fori_loop(0, cnt - ngrp * 2 * GG16, tailb, 0)

        @pl.when(cid == 0)
        def _():
            _zero_my_rows(acc0, zbuf, tid)
            _zero_my_rows(acc1, zbuf, tid)
            plsc.subcore_barrier()
            hist(dc_h, acc0)
            hist(dcb_h, acc1)
            plsc.subcore_barrier()
            _copy_my_rows(acc0, qc_h, tid)
            _copy_my_rows(acc1, qcb_h, tid)

        @pl.when(cid == 1)
        def _():
            _zero_my_rows(acc0, zbuf, tid)
            plsc.subcore_barrier()
            hist(df_h, acc0)
            plsc.subcore_barrier()
            _copy_my_rows(acc0, qf_h, tid)

    return k(dst_c, dst_cb, dst_f)


RB = 1000
_GRID = N // RB


def _lrelu(x):
    return jnp.where(x >= 0, x, 0.01 * x)


def _qgate(cnt_r):
    c = cnt_r[...][:, 0:1]
    q = jnp.where(c > 0.5, 1.0 / jnp.maximum(c, 1.0), 0.0)
    return q, (c > 0.5).astype(jnp.float32)


def _chunk_mm(s, w):
    acc = jnp.dot(s[0], w[0:CCH, :], preferred_element_type=jnp.float32)
    for c in range(1, NCH):
        acc = acc + jnp.dot(s[c], w[c * CCH:(c + 1) * CCH, :],
                            preferred_element_type=jnp.float32)
    return acc


def _proj(s_r, q_r, w0_r, b0_r, w1_r, wlin):
    q, g = _qgate(q_r)
    x0 = q * _chunk_mm(s_r[...], w0_r[...]) + g * b0_r[...]
    return jnp.dot(_lrelu(x0),
                   jnp.dot(w1_r[...], wlin, preferred_element_type=jnp.float32),
                   preferred_element_type=jnp.float32)


def _stage_b1_body(sc_r, qc_r, w0c_r, b0c_r, w1cb_r, wlin_r, zi_r):
    zi_r[...] = _proj(sc_r, qc_r, w0c_r, b0c_r, w1cb_r, wlin_r[...])


def _stage_b2_body(scb_r, qcb_r, sf_r, qf_r, w0cb_r, b0cb_r, w0f_r, b0f_r,
                   w1f_r, wlin_r, zu_r):
    wlin = wlin_r[...]
    qcb, gcb = _qgate(qcb_r)
    qf, gf = _qgate(qf_r)
    user0 = (qcb * _chunk_mm(scb_r[...], w0cb_r[...]) + gcb * b0cb_r[...]
             + qf * _chunk_mm(sf_r[...], w0f_r[...]) + gf * b0f_r[...])
    zu_r[...] = jnp.dot(_lrelu(user0),
                        jnp.dot(w1f_r[...], wlin,
                                preferred_element_type=jnp.float32),
                        preferred_element_type=jnp.float32)


_S_SPEC = lambda: pl.BlockSpec((NCH, RB, CCH), lambda r: (0, r, 0))
_Q_SPEC = lambda: pl.BlockSpec((RB, OUT), lambda r: (r, 0))
_W_SPEC = lambda: pl.BlockSpec((HID, HID), lambda r: (0, 0))
_B_SPEC = lambda: pl.BlockSpec((1, HID), lambda r: (0, 0))
_L_SPEC = lambda: pl.BlockSpec((HID, OUT), lambda r: (0, 0))


def _stage_b1(sc, qc, w0c, b0c, w1cb, wlin):
    return pl.pallas_call(
        _stage_b1_body,
        grid=(_GRID,),
        in_specs=[_S_SPEC(), _Q_SPEC(), _W_SPEC(), _B_SPEC(), _W_SPEC(),
                  _L_SPEC()],
        out_specs=_Q_SPEC(),
        out_shape=jax.ShapeDtypeStruct((N, OUT), jnp.float32),
    )(sc, qc, w0c, b0c, w1cb, wlin)


def _stage_b2(scb, qcb, sf, qf, w0cb, b0cb, w0f, b0f, w1f, wlin):
    return pl.pallas_call(
        _stage_b2_body,
        grid=(_GRID,),
        in_specs=[_S_SPEC(), _Q_SPEC(), _S_SPEC(), _Q_SPEC(), _W_SPEC(),
                  _B_SPEC(), _W_SPEC(), _B_SPEC(), _W_SPEC(), _L_SPEC()],
        out_specs=_Q_SPEC(),
        out_shape=jax.ShapeDtypeStruct((N, OUT), jnp.float32),
    )(scb, qcb, sf, qf, w0cb, b0cb, w0f, b0f, w1f, wlin)


def _stage_d_body(p0_r, p1_r, p2_r, p3_r, qcb_r, qf_r, b1cb_r, b1f_r,
                  wlin_r, linb_r, out_r):
    qcb, gcb = _qgate(qcb_r)
    qf, gf = _qgate(qf_r)
    wlin = wlin_r[...]
    bias = (gcb * jnp.dot(b1cb_r[...], wlin, preferred_element_type=jnp.float32)
            + gf * jnp.dot(b1f_r[...], wlin, preferred_element_type=jnp.float32)
            + linb_r[...])
    out_r[...] = (qcb * (p0_r[...] + p1_r[...])
                  + qf * (p2_r[...] + p3_r[...]) + bias)


def _stage_d(p0, p1, p2, p3, qcb, qf, b1cb, b1f, wlin, linb):
    return pl.pallas_call(
        _stage_d_body,
        grid=(_GRID,),
        in_specs=[_Q_SPEC(), _Q_SPEC(), _Q_SPEC(), _Q_SPEC(), _Q_SPEC(),
                  _Q_SPEC(), _B_SPEC(), _B_SPEC(), _L_SPEC(),
                  pl.BlockSpec((1, OUT), lambda r: (0, 0))],
        out_specs=_Q_SPEC(),
        out_shape=jax.ShapeDtypeStruct((N, OUT), jnp.float32),
    )(p0, p1, p2, p3, qcb, qf, b1cb, b1f, wlin, linb)


def kernel(features, embed_item, edge_index_clicks, edge_index_clicked_by,
           edge_index_follows, W0_clicks, b0_clicks, W0_clicked_by,
           b0_clicked_by, W0_follows, b0_follows, W1_clicks, b1_clicks,
           W1_clicked_by, b1_clicked_by, W1_follows, b1_follows, lin_W,
           lin_b):
    i32 = jnp.int32
    r2 = lambda x: x.astype(i32).reshape(NBLK, BLK)
    sc_, dc_ = r2(edge_index_clicks[0]), r2(edge_index_clicks[1])
    scb, dcb = r2(edge_index_clicked_by[0]), r2(edge_index_clicked_by[1])
    sf_, df_ = r2(edge_index_follows[0]), r2(edge_index_follows[1])

    f4 = features.reshape(N, NCH, CCH).transpose(1, 0, 2)
    e4 = embed_item.reshape(N, NCH, CCH).transpose(1, 0, 2)

    qc, qcb, qf = _counts(dc_, dcb, df_)
    s_clicks = _seg128(f4, sc_, dc_)
    s_cb = _seg128(e4, scb, dcb)
    s_f = _seg128(f4, sf_, df_)

    zi = _stage_b1(s_clicks, qc, W0_clicks, b0_clicks.reshape(1, HID),
                   W1_clicked_by, lin_W)
    zu = _stage_b2(s_cb, qcb, s_f, qf,
                   W0_clicked_by, b0_clicked_by.reshape(1, HID),
                   W0_follows, b0_follows.reshape(1, HID),
                   W1_follows, lin_W)

    p0, p1, p2, p3 = _seg16(zi, zu, scb, dcb, sf_, df_)

    return _stage_d(p0, p1, p2, p3, qcb, qf, b1_clicked_by.reshape(1, HID),
                    b1_follows.reshape(1, HID), lin_W,
                    lin_b.reshape(1, OUT))


# speedup vs baseline: 1.3226x; 1.1183x over previous
import functools

import jax
import jax.numpy as jnp
from jax import lax
from jax.experimental import pallas as pl
from jax.experimental.pallas import tpu as pltpu
from jax.experimental.pallas import tpu_sc as plsc

N = 50000
E = 400000
HID = 128
OUT = 16
NSUB = 16
NCORE = 2
BLK = 128
NBLK = E // BLK
CCH = 32
NCH = HID // CCH
ZR = 80
RPT = 3200
GG = 3
GG16 = 8
NU = 4

_MESH = plsc.VectorSubcoreMesh(core_axis_name="c", subcore_axis_name="s")


def _span(total, parts, i):
    base = total // parts
    rem = total % parts
    start = i * base + jnp.minimum(i, rem)
    cnt = base + jnp.where(i < rem, 1, 0).astype(jnp.int32)
    return start, cnt


def _fill(buf, rows, cols, value):
    v = jnp.full((16,), value, jnp.float32)

    def row(i, _):
        for c0 in range(0, cols, 16):
            buf[i, pl.ds(c0, 16)] = v
        return 0

    lax.fori_loop(0, rows, row, 0)


def _per_tile_rows(tid, fn):
    @pl.when(tid < NSUB - 1)
    def _():
        fn(tid * RPT, RPT // ZR)

    @pl.when(tid == NSUB - 1)
    def _():
        fn((NSUB - 1) * RPT, (N - (NSUB - 1) * RPT) // ZR)


def _zero_my_rows(acc_sp, z_h, tid):
    def fn(row0, nch):
        pltpu.sync_copy(z_h.at[pl.ds(0, nch * ZR)],
                        acc_sp.at[pl.ds(row0, nch * ZR)])

    _per_tile_rows(tid, fn)


def _copy_my_rows(acc_sp, out_ref, tid):
    def fn(row0, nch):
        pltpu.sync_copy(acc_sp.at[pl.ds(row0, nch * ZR)],
                        out_ref.at[pl.ds(row0, nch * ZR)])

    _per_tile_rows(tid, fn)


def _accumulate(gg, tab_h, src2_h, dst2_h, acc_sp, sg, dg, rb0, rb1, gsem0,
                gsem1, ssem0, ssem1, start, cnt):
    rbs = (rb0, rb1)
    gsems = (gsem0, gsem1)
    ssems = (ssem0, ssem1)

    def fire_gather(u):
        rb = rbs[u % 2]
        return [pltpu.async_copy(tab_h.at[sg.at[u * gg + j]],
                                 rb.at[pl.ds(j * BLK, BLK)], gsems[u % 2])
                for j in range(gg)]

    def fire_scatter(u):
        rb = rbs[u % 2]
        return [pltpu.async_copy(rb.at[pl.ds(j * BLK, BLK)],
                                 acc_sp.at[dg.at[u * gg + j]], ssems[u % 2],
                                 add=True)
                for j in range(gg)]

    nsup = cnt // (NU * gg)

    def sup(s, _):
        base = start + s * NU * gg
        pltpu.sync_copy(src2_h.at[pl.ds(base, NU * gg)], sg)
        pltpu.sync_copy(dst2_h.at[pl.ds(base, NU * gg)], dg)
        g = [fire_gather(0), fire_gather(1)] + [None] * (NU - 2)
        sc = [None] * NU
        for u in range(NU):
            for d in g[u]:
                d.wait()
            sc[u] = fire_scatter(u)
            if u >= 1 and u + 1 < NU:
                for d in sc[u - 1]:
                    d.wait()
                g[u + 1] = fire_gather(u + 1)
        for d in sc[NU - 2]:
            d.wait()
        for d in sc[NU - 1]:
            d.wait()
        return 0

    lax.fori_loop(0, nsup, sup, 0)

    def tail(t, _):
        b = start + nsup * NU * gg + t
        pltpu.sync_copy(src2_h.at[pl.ds(b, 1)], sg.at[pl.ds(0, 1)])
        pltpu.sync_copy(dst2_h.at[pl.ds(b, 1)], dg.at[pl.ds(0, 1)])
        pltpu.sync_copy(tab_h.at[sg.at[0]], rb0.at[pl.ds(0, BLK)])
        pltpu.sync_copy(rb0.at[pl.ds(0, BLK)], acc_sp.at[dg.at[0]], add=True)
        return 0

    lax.fori_loop(0, cnt - nsup * NU * gg, tail, 0)


def _seg128(t4, src, dst, zeros):
    @functools.partial(
        pl.kernel,
        out_type=jax.ShapeDtypeStruct((NCH, N, CCH), jnp.float32),
        mesh=_MESH,
        compiler_params=pltpu.CompilerParams(use_tc_tiling_on_sc=False),
        scratch_types=[
            pltpu.VMEM_SHARED((N, CCH), jnp.float32),
            pltpu.VMEM((NU * GG, BLK), jnp.int32),
            pltpu.VMEM((NU * GG, BLK), jnp.int32),
            pltpu.VMEM((GG * BLK, CCH), jnp.float32),
            pltpu.VMEM((GG * BLK, CCH), jnp.float32),
            pltpu.SemaphoreType.DMA,
            pltpu.SemaphoreType.DMA,
            pltpu.SemaphoreType.DMA,
            pltpu.SemaphoreType.DMA,
        ],
    )
    def k(t4_h, src_h, dst_h, z_h, out_h, acc_sp, sg, dg, rb0, rb1, gsem0,
          gsem1, ssem0, ssem1):
        cid = lax.axis_index("c")
        tid = lax.axis_index("s")
        start, cnt = _span(NBLK, NSUB, tid)

        def do_chunk(ci):
            _zero_my_rows(acc_sp, z_h, tid)
            plsc.subcore_barrier()
            _accumulate(GG, t4_h.at[ci], src_h, dst_h, acc_sp, sg, dg, rb0,
                        rb1, gsem0, gsem1, ssem0, ssem1, start, cnt)
            plsc.subcore_barrier()
            _copy_my_rows(acc_sp, out_h.at[ci], tid)
            plsc.subcore_barrier()

        for j in range(NCH // NCORE):
            @pl.when(cid == 0)
            def _():
                do_chunk(j)

            @pl.when(cid == 1)
            def _():
                do_chunk(NCH // NCORE + j)

    return k(t4, src, dst, zeros)


def _seg16(z0, z1, src0, dst0, src1, dst1, zeros):
    @functools.partial(
        pl.kernel,
        out_type=(jax.ShapeDtypeStruct((N, OUT), jnp.float32),) * 4,
        mesh=_MESH,
        compiler_params=pltpu.CompilerParams(use_tc_tiling_on_sc=False),
        scratch_types=[
            pltpu.VMEM_SHARED((N, OUT), jnp.float32),
            pltpu.VMEM((NU * GG16, BLK), jnp.int32),
            pltpu.VMEM((NU * GG16, BLK), jnp.int32),
            pltpu.VMEM((GG16 * BLK, OUT), jnp.float32),
            pltpu.VMEM((GG16 * BLK, OUT), jnp.float32),
            pltpu.SemaphoreType.DMA,
            pltpu.SemaphoreType.DMA,
            pltpu.SemaphoreType.DMA,
            pltpu.SemaphoreType.DMA,
        ],
    )
    def k(z0_h, z1_h, s0_h, d0_h, s1_h, d1_h, zz_h, p0_h, p1_h, p2_h, p3_h,
          acc_sp, sg, dg, rb0, rb1, gsem0, gsem1, ssem0, ssem1):
        cid = lax.axis_index("c")
        tid = lax.axis_index("s")
        cstart, ccnt = _span(NBLK, NCORE, cid)
        tstart, tcnt = _span(ccnt, NSUB, tid)
        start = cstart + tstart

        for rel, (z_h, s_h, d_h) in enumerate(((z0_h, s0_h, d0_h),
                                               (z1_h, s1_h, d1_h))):
            _zero_my_rows(acc_sp, zz_h, tid)
            plsc.subcore_barrier()
            _accumulate(GG16, z_h, s_h, d_h, acc_sp, sg, dg, rb0, rb1,
                        gsem0, gsem1, ssem0, ssem1, start, tcnt)
            plsc.subcore_barrier()

            outs = ((p0_h, p1_h), (p2_h, p3_h))[rel]

            @pl.when(cid == 0)
            def _():
                _copy_my_rows(acc_sp, outs[0], tid)

            @pl.when(cid == 1)
            def _():
                _copy_my_rows(acc_sp, outs[1], tid)

            plsc.subcore_barrier()

    return k(z0, z1, src0, dst0, src1, dst1, zeros)


def _counts(dst_c, dst_cb, dst_f, zeros):
    @functools.partial(
        pl.kernel,
        out_type=(jax.ShapeDtypeStruct((N, OUT), jnp.float32),) * 3,
        mesh=_MESH,
        compiler_params=pltpu.CompilerParams(use_tc_tiling_on_sc=False),
        scratch_types=[
            pltpu.VMEM_SHARED((N, OUT), jnp.float32),
            pltpu.VMEM_SHARED((N, OUT), jnp.float32),
            pltpu.VMEM((BLK, OUT), jnp.float32),
            pltpu.VMEM((2 * GG16, BLK), jnp.int32),
            pltpu.SemaphoreType.DMA,
        ],
    )
    def k(dc_h, dcb_h, df_h, z_h, qc_h, qcb_h, qf_h, acc0, acc1, ones, dbuf,
          ssem):
        cid = lax.axis_index("c")
        tid = lax.axis_index("s")
        _fill(ones, BLK, OUT, 1.0)
        start, cnt = _span(NBLK, NSUB, tid)

        def hist(d_h, acc):
            def group(base, nb):
                pltpu.sync_copy(d_h.at[pl.ds(base, nb)],
                                dbuf.at[pl.ds(0, nb)])
                sds = [pltpu.async_copy(ones, acc.at[dbuf.at[j]], ssem,
                                        add=True) for j in range(nb)]
                for d in sds:
                    d.wait()

            ngrp = cnt // (2 * GG16)

            def body(g, _):
                group(start + g * 2 * GG16, 2 * GG16)
                return 0

            lax.fori_loop(0, ngrp, body, 0)

            def tailb(t, _):
                group(start + ngrp * 2 * GG16 + t, 1)
                return 0

            lax.fori_loop(0, cnt - ngrp * 2 * GG16, tailb, 0)

        @pl.when(cid == 0)
        def _():
            _zero_my_rows(acc0, z_h, tid)
            _zero_my_rows(acc1, z_h, tid)
            plsc.subcore_barrier()
            hist(dc_h, acc0)
            hist(dcb_h, acc1)
            plsc.subcore_barrier()
            _copy_my_rows(acc0, qc_h, tid)
            _copy_my_rows(acc1, qcb_h, tid)

        @pl.when(cid == 1)
        def _():
            _zero_my_rows(acc0, z_h, tid)
            plsc.subcore_barrier()
            hist(df_h, acc0)
            plsc.subcore_barrier()
            _copy_my_rows(acc0, qf_h, tid)

    return k(dst_c, dst_cb, dst_f, zeros)


RB = 1000
_GRID = N // RB


def _lrelu(x):
    return jnp.where(x >= 0, x, 0.01 * x)


def _qgate(cnt_r):
    c = cnt_r[...][:, 0:1]
    q = jnp.where(c > 0.5, 1.0 / jnp.maximum(c, 1.0), 0.0)
    return q, (c > 0.5).astype(jnp.float32)


def _chunk_mm(s, w):
    acc = jnp.dot(s[0], w[0:CCH, :], preferred_element_type=jnp.float32)
    for c in range(1, NCH):
        acc = acc + jnp.dot(s[c], w[c * CCH:(c + 1) * CCH, :],
                            preferred_element_type=jnp.float32)
    return acc


def _proj(s_r, q_r, w0_r, b0_r, w1_r, wlin):
    q, g = _qgate(q_r)
    x0 = q * _chunk_mm(s_r[...], w0_r[...]) + g * b0_r[...]
    return jnp.dot(_lrelu(x0),
                   jnp.dot(w1_r[...], wlin, preferred_element_type=jnp.float32),
                   preferred_element_type=jnp.float32)


def _stage_b1_body(sc_r, qc_r, w0c_r, b0c_r, w1cb_r, wlin_r, zi_r):
    zi_r[...] = _proj(sc_r, qc_r, w0c_r, b0c_r, w1cb_r, wlin_r[...])


def _stage_b2_body(scb_r, qcb_r, sf_r, qf_r, w0cb_r, b0cb_r, w0f_r, b0f_r,
                   w1f_r, wlin_r, zu_r):
    wlin = wlin_r[...]
    qcb, gcb = _qgate(qcb_r)
    qf, gf = _qgate(qf_r)
    user0 = (qcb * _chunk_mm(scb_r[...], w0cb_r[...]) + gcb * b0cb_r[...]
             + qf * _chunk_mm(sf_r[...], w0f_r[...]) + gf * b0f_r[...])
    zu_r[...] = jnp.dot(_lrelu(user0),
                        jnp.dot(w1f_r[...], wlin,
                                preferred_element_type=jnp.float32),
                        preferred_element_type=jnp.float32)


_S_SPEC = lambda: pl.BlockSpec((NCH, RB, CCH), lambda r: (0, r, 0))
_Q_SPEC = lambda: pl.BlockSpec((RB, OUT), lambda r: (r, 0))
_W_SPEC = lambda: pl.BlockSpec((HID, HID), lambda r: (0, 0))
_B_SPEC = lambda: pl.BlockSpec((1, HID), lambda r: (0, 0))
_L_SPEC = lambda: pl.BlockSpec((HID, OUT), lambda r: (0, 0))


def _stage_b1(sc, qc, w0c, b0c, w1cb, wlin):
    return pl.pallas_call(
        _stage_b1_body,
        grid=(_GRID,),
        in_specs=[_S_SPEC(), _Q_SPEC(), _W_SPEC(), _B_SPEC(), _W_SPEC(),
                  _L_SPEC()],
        out_specs=_Q_SPEC(),
        out_shape=jax.ShapeDtypeStruct((N, OUT), jnp.float32),
    )(sc, qc, w0c, b0c, w1cb, wlin)


def _stage_b2(scb, qcb, sf, qf, w0cb, b0cb, w0f, b0f, w1f, wlin):
    return pl.pallas_call(
        _stage_b2_body,
        grid=(_GRID,),
        in_specs=[_S_SPEC(), _Q_SPEC(), _S_SPEC(), _Q_SPEC(), _W_SPEC(),
                  _B_SPEC(), _W_SPEC(), _B_SPEC(), _W_SPEC(), _L_SPEC()],
        out_specs=_Q_SPEC(),
        out_shape=jax.ShapeDtypeStruct((N, OUT), jnp.float32),
    )(scb, qcb, sf, qf, w0cb, b0cb, w0f, b0f, w1f, wlin)


def _stage_d_body(p0_r, p1_r, p2_r, p3_r, qcb_r, qf_r, b1cb_r, b1f_r,
                  wlin_r, linb_r, out_r):
    qcb, gcb = _qgate(qcb_r)
    qf, gf = _qgate(qf_r)
    wlin = wlin_r[...]
    bias = (gcb * jnp.dot(b1cb_r[...], wlin, preferred_element_type=jnp.float32)
            + gf * jnp.dot(b1f_r[...], wlin, preferred_element_type=jnp.float32)
            + linb_r[...])
    out_r[...] = (qcb * (p0_r[...] + p1_r[...])
                  + qf * (p2_r[...] + p3_r[...]) + bias)


def _stage_d(p0, p1, p2, p3, qcb, qf, b1cb, b1f, wlin, linb):
    return pl.pallas_call(
        _stage_d_body,
        grid=(_GRID,),
        in_specs=[_Q_SPEC(), _Q_SPEC(), _Q_SPEC(), _Q_SPEC(), _Q_SPEC(),
                  _Q_SPEC(), _B_SPEC(), _B_SPEC(), _L_SPEC(),
                  pl.BlockSpec((1, OUT), lambda r: (0, 0))],
        out_specs=_Q_SPEC(),
        out_shape=jax.ShapeDtypeStruct((N, OUT), jnp.float32),
    )(p0, p1, p2, p3, qcb, qf, b1cb, b1f, wlin, linb)


def kernel(features, embed_item, edge_index_clicks, edge_index_clicked_by,
           edge_index_follows, W0_clicks, b0_clicks, W0_clicked_by,
           b0_clicked_by, W0_follows, b0_follows, W1_clicks, b1_clicks,
           W1_clicked_by, b1_clicked_by, W1_follows, b1_follows, lin_W,
           lin_b):
    i32 = jnp.int32
    r2 = lambda x: x.astype(i32).reshape(NBLK, BLK)
    sc_, dc_ = r2(edge_index_clicks[0]), r2(edge_index_clicks[1])
    scb, dcb = r2(edge_index_clicked_by[0]), r2(edge_index_clicked_by[1])
    sf_, df_ = r2(edge_index_follows[0]), r2(edge_index_follows[1])

    f4 = features.reshape(N, NCH, CCH).transpose(1, 0, 2)
    e4 = embed_item.reshape(N, NCH, CCH).transpose(1, 0, 2)

    z32 = jnp.zeros((RPT, CCH), jnp.float32)
    z16 = jnp.zeros((RPT, OUT), jnp.float32)

    qc, qcb, qf = _counts(dc_, dcb, df_, z16)
    s_clicks = _seg128(f4, sc_, dc_, z32)
    s_cb = _seg128(e4, scb, dcb, z32)
    s_f = _seg128(f4, sf_, df_, z32)

    zi = _stage_b1(s_clicks, qc, W0_clicks, b0_clicks.reshape(1, HID),
                   W1_clicked_by, lin_W)
    zu = _stage_b2(s_cb, qcb, s_f, qf,
                   W0_clicked_by, b0_clicked_by.reshape(1, HID),
                   W0_follows, b0_follows.reshape(1, HID),
                   W1_follows, lin_W)

    p0, p1, p2, p3 = _seg16(zi, zu, scb, dcb, sf_, df_, z16)

    return _stage_d(p0, p1, p2, p3, qcb, qf, b1_clicked_by.reshape(1, HID),
                    b1_follows.reshape(1, HID), lin_W,
                    lin_b.reshape(1, OUT))
